# Initial kernel scaffold; baseline (speedup 1.0000x reference)
import functools

import jax
import jax.numpy as jnp
from jax import lax
from jax.experimental import pallas as pl
from jax.experimental.pallas import tpu as pltpu
from jax.experimental.pallas import tpu_sc as plsc

N = 10000
E = 160000
NFEAT = 256
NHID = 64
NHEADS = 4
NCLASS = 40
NEG_SLOPE = 0.2

NP = 10240
NC = 2
NS = 16
NW = NC * NS
K = 128
NCHUNK = 40
EPW = K * NCHUNK
EPAD = NW * EPW
ACOL = 80
ROWS_PER_TILE = NP // NS
ZR = 16

BN = 256
GRID = NP // BN


def _elu(v):
    return jnp.where(v > 0.0, v, jnp.expm1(v))


def _norm_cols(acc_refs, nheads, double_elu):
    cols = []
    for h in range(nheads):
        pa = acc_refs[h][0] + acc_refs[h][1]
        o = pa[:, :NHID] / (pa[:, NHID:NHID + 1] + 1e-16)
        o = _elu(o)
        if double_elu:
            o = _elu(o)
        cols.append(o)
    return jnp.concatenate(cols, axis=1) if len(cols) > 1 else cols[0]


def _heads_matmul(xb, w_ref, as_ref, ad_ref, nheads, ht_refs, sd_ref):
    scols = []
    dcols = []
    for h in range(nheads):
        hb = jnp.dot(xb, w_ref[h], preferred_element_type=jnp.float32)
        ht_refs[h][...] = hb
        scols.append(jnp.dot(hb, as_ref[h].reshape(NHID, 1),
                             preferred_element_type=jnp.float32))
        dcols.append(jnp.dot(hb, ad_ref[h].reshape(NHID, 1),
                             preferred_element_type=jnp.float32))
    sd_ref[...] = jnp.concatenate(scols + dcols, axis=1)


def _make_tc1(interpret=False):
    def body(x_ref, w_ref, as_ref, ad_ref, ht0, ht1, ht2, ht3, sd_ref):
        _heads_matmul(x_ref[...], w_ref, as_ref, ad_ref, NHEADS,
                      (ht0, ht1, ht2, ht3), sd_ref)

    full = lambda shape: pl.BlockSpec(shape, lambda i: (0,) * len(shape))
    return pl.pallas_call(
        body,
        grid=(GRID,),
        in_specs=[
            pl.BlockSpec((BN, NFEAT), lambda i: (i, 0)),
            full((NHEADS, NFEAT, NHID)),
            full((NHEADS, NHID)),
            full((NHEADS, NHID)),
        ],
        out_specs=[pl.BlockSpec((BN, NHID), lambda i: (i, 0))] * NHEADS
        + [pl.BlockSpec((BN, 2 * NHEADS), lambda i: (i, 0))],
        out_shape=[jax.ShapeDtypeStruct((NP, NHID), jnp.float32)] * NHEADS
        + [jax.ShapeDtypeStruct((NP, 2 * NHEADS), jnp.float32)],
        interpret=interpret,
    )


def _make_tc_mid(double_elu, interpret=False):
    def body(a0, a1, a2, a3, w_ref, as_ref, ad_ref, ht0, ht1, ht2, ht3, sd_ref):
        hcat = _norm_cols((a0, a1, a2, a3), NHEADS, double_elu)
        _heads_matmul(hcat, w_ref, as_ref, ad_ref, NHEADS,
                      (ht0, ht1, ht2, ht3), sd_ref)

    full = lambda shape: pl.BlockSpec(shape, lambda i: (0,) * len(shape))
    return pl.pallas_call(
        body,
        grid=(GRID,),
        in_specs=[pl.BlockSpec((NC, BN, ACOL), lambda i: (0, i, 0))] * NHEADS
        + [
            full((NHEADS, NFEAT, NHID)),
            full((NHEADS, NHID)),
            full((NHEADS, NHID)),
        ],
        out_specs=[pl.BlockSpec((BN, NHID), lambda i: (i, 0))] * NHEADS
        + [pl.BlockSpec((BN, 2 * NHEADS), lambda i: (i, 0))],
        out_shape=[jax.ShapeDtypeStruct((NP, NHID), jnp.float32)] * NHEADS
        + [jax.ShapeDtypeStruct((NP, 2 * NHEADS), jnp.float32)],
        interpret=interpret,
    )


def _make_tc3(interpret=False):
    def body(a0, a1, a2, a3, w_ref, as_ref, ad_ref, ht_ref, sd_ref):
        hcat = _norm_cols((a0, a1, a2, a3), NHEADS, False)
        hb = jnp.dot(hcat, w_ref[...], preferred_element_type=jnp.float32)
        ht_ref[...] = hb
        s = jnp.dot(hb, as_ref[...].reshape(NHID, 1),
                    preferred_element_type=jnp.float32)
        d = jnp.dot(hb, ad_ref[...].reshape(NHID, 1),
                    preferred_element_type=jnp.float32)
        sd_ref[...] = jnp.concatenate([s, d], axis=1)

    full = lambda shape: pl.BlockSpec(shape, lambda i: (0,) * len(shape))
    return pl.pallas_call(
        body,
        grid=(GRID,),
        in_specs=[pl.BlockSpec((NC, BN, ACOL), lambda i: (0, i, 0))] * NHEADS
        + [full((NFEAT, NHID)), full((NHID,)), full((NHID,))],
        out_specs=[
            pl.BlockSpec((BN, NHID), lambda i: (i, 0)),
            pl.BlockSpec((BN, 2), lambda i: (i, 0)),
        ],
        out_shape=[
            jax.ShapeDtypeStruct((NP, NHID), jnp.float32),
            jax.ShapeDtypeStruct((NP, 2), jnp.float32),
        ],
        interpret=interpret,
    )


def _make_tc4(interpret=False):
    def body(a_ref, o_ref):
        pa = a_ref[0] + a_ref[1]
        o = pa[:, :NHID] / (pa[:, NHID:NHID + 1] + 1e-16)
        o_ref[...] = _elu(o)

    return pl.pallas_call(
        body,
        grid=(GRID,),
        in_specs=[pl.BlockSpec((NC, BN, ACOL), lambda i: (0, i, 0))],
        out_specs=pl.BlockSpec((BN, NHID), lambda i: (i, 0)),
        out_shape=jax.ShapeDtypeStruct((NP, NHID), jnp.float32),
        interpret=interpret,
    )


def _make_sc(nheads, interpret=False):
    mesh = plsc.VectorSubcoreMesh(core_axis_name="c", subcore_axis_name="s",
                                  num_cores=NC, num_subcores=NS)
    out_type = [jax.ShapeDtypeStruct((NC, NP, ACOL), jnp.float32)
                for _ in range(nheads)]
    scratch = [
        pltpu.VMEM((NCHUNK, K), jnp.int32),
        pltpu.VMEM((NCHUNK, K), jnp.int32),
        pltpu.VMEM((NP, 2 * nheads), jnp.float32),
        pltpu.VMEM((K,), jnp.float32),
        pltpu.VMEM((K, NHID), jnp.float32),
        pltpu.VMEM((K, ACOL), jnp.float32),
        pltpu.VMEM((ZR, ACOL), jnp.float32),
        pltpu.VMEM_SHARED((NP, ACOL), jnp.float32),
        pltpu.SemaphoreType.DMA,
    ]

    @functools.partial(pl.kernel, out_type=out_type, mesh=mesh,
                       scratch_types=scratch, interpret=interpret)
    def sck(src_hbm, dst_hbm, sd_hbm, *rest):
        ht = rest[:nheads]
        outs = rest[nheads:2 * nheads]
        (src_v, dst_v, sd_v, ex_v, rows_v, st_v, zeros_v, acc_sh,
         sem) = rest[2 * nheads:]
        cid = lax.axis_index("c")
        sid = lax.axis_index("s")
        wid = cid * NS + sid
        pltpu.sync_copy(src_hbm.at[wid], src_v)
        pltpu.sync_copy(dst_hbm.at[wid], dst_v)
        pltpu.sync_copy(sd_hbm, sd_v)
        z16 = jnp.zeros((16,), jnp.float32)
        for r in range(ZR):
            for q in range(ACOL // 16):
                zeros_v[r, pl.ds(q * 16, 16)] = z16
        iota16 = lax.iota(jnp.int32, 16)
        row0 = sid * ROWS_PER_TILE

        for h in range(nheads):
            def zbody(i, _):
                pltpu.sync_copy(zeros_v, acc_sh.at[pl.ds(row0 + i * ZR, ZR)])
                return 0
            lax.fori_loop(0, ROWS_PER_TILE // ZR, zbody, 0)
            plsc.subcore_barrier()

            col_s = jnp.full((16,), h, jnp.int32)
            col_d = jnp.full((16,), nheads + h, jnp.int32)

            def chunk_body(c, _):
                for j in range(K // 16):
                    si = src_v[c, pl.ds(j * 16, 16)]
                    di = dst_v[c, pl.ds(j * 16, 16)]
                    sv = plsc.load_gather(sd_v, [si, col_s])
                    dv = plsc.load_gather(sd_v, [di, col_d])
                    e = sv + dv
                    e = jnp.where(e >= 0.0, e, e * NEG_SLOPE)
                    ex_v[pl.ds(j * 16, 16)] = jnp.exp(e)
                pltpu.async_copy(ht[h].at[src_v.at[c]], rows_v, sem).wait()

                def sbody(j, _):
                    a = ex_v[j]
                    av = jnp.full((16,), a, jnp.float32)
                    for q in range(NHID // 16):
                        st_v[j, pl.ds(q * 16, 16)] = (
                            rows_v[j, pl.ds(q * 16, 16)] * av)
                    st_v[j, pl.ds(NHID, 16)] = jnp.where(
                        iota16 == 0, av, 0.0)
                    return 0
                lax.fori_loop(0, K, sbody, 0)
                pltpu.sync_copy(st_v, acc_sh.at[dst_v.at[c]], add=True)
                return 0
            lax.fori_loop(0, NCHUNK, chunk_body, 0)
            plsc.subcore_barrier()
            pltpu.sync_copy(
                acc_sh.at[pl.ds(row0, ROWS_PER_TILE)],
                outs[h].at[cid, pl.ds(row0, ROWS_PER_TILE)])
            plsc.subcore_barrier()

    return sck


_tc1 = _make_tc1()
_tc2 = _make_tc_mid(double_elu=True)
_tc3 = _make_tc3()
_tc4 = _make_tc4()
_sc4 = _make_sc(NHEADS)
_sc1 = _make_sc(1)


@jax.jit
def _impl(x, edge_index, W1, a1_src, a1_dst, W2, a2_src, a2_dst,
          W3, a3_src, a3_dst):
    f32 = jnp.float32
    xp = jnp.zeros((NP, NFEAT), f32).at[:N].set(x.astype(f32))
    src = edge_index[0].astype(jnp.int32)
    dst = edge_index[1].astype(jnp.int32)
    pad = jnp.full((EPAD - E,), NP - 1, jnp.int32)
    src2 = jnp.concatenate([src, pad]).reshape(NW, NCHUNK, K)
    dst2 = jnp.concatenate([dst, pad]).reshape(NW, NCHUNK, K)
    W3p = jnp.zeros((NFEAT, NHID), f32).at[:, :NCLASS].set(W3)
    a3sp = jnp.zeros((NHID,), f32).at[:NCLASS].set(a3_src)
    a3dp = jnp.zeros((NHID,), f32).at[:NCLASS].set(a3_dst)

    *ht1, sd1 = _tc1(xp, W1, a1_src, a1_dst)
    acc1 = _sc4(src2, dst2, sd1, *ht1)
    *ht2, sd2 = _tc2(*acc1, W2, a2_src, a2_dst)
    acc2 = _sc4(src2, dst2, sd2, *ht2)
    ht3, sd3 = _tc3(*acc2, W3p, a3sp, a3dp)
    acc3 = _sc1(src2, dst2, sd3, ht3)
    out = _tc4(acc3[0])
    return out[:N, :NCLASS]


def kernel(x, edge_index, n_node_features, mini_batch,
           W1, a1_src, a1_dst, W2, a2_src, a2_dst, W3, a3_src, a3_dst):
    return _impl(x, edge_index, W1, a1_src, a1_dst,
                 W2, a2_src, a2_dst, W3, a3_src, a3_dst)


# speedup vs baseline: 15.8988x; 15.8988x over previous
import functools

import jax
import jax.numpy as jnp
from jax import lax
from jax.experimental import pallas as pl
from jax.experimental.pallas import tpu as pltpu
from jax.experimental.pallas import tpu_sc as plsc

N = 10000
E = 160000
NFEAT = 256
NHID = 64
NHEADS = 4
NCLASS = 40
NEG_SLOPE = 0.2

NP = 10240
NC = 2
NS = 16
NW = NC * NS
K = 128
NCHUNK = 40
EPW = K * NCHUNK
EPAD = NW * EPW
ACOL = 80
ROWS_PER_TILE = NP // NS
ZR = 16

BN = 256
GRID = NP // BN


def _elu(v):
    return jnp.where(v > 0.0, v, jnp.exp(jnp.minimum(v, 0.0)) - 1.0)


def _norm_cols(acc_refs, nheads, double_elu):
    cols = []
    for h in range(nheads):
        pa = acc_refs[h][0] + acc_refs[h][1]
        o = pa[:, :NHID] / (pa[:, NHID:NHID + 1] + 1e-16)
        o = _elu(o)
        if double_elu:
            o = _elu(o)
        cols.append(o)
    return jnp.concatenate(cols, axis=1) if len(cols) > 1 else cols[0]


def _heads_matmul(xb, w_ref, as_ref, ad_ref, nheads, ht_refs, sd_ref):
    scols = []
    dcols = []
    for h in range(nheads):
        hb = jnp.dot(xb, w_ref[h], preferred_element_type=jnp.float32)
        ht_refs[h][...] = hb
        scols.append(jnp.dot(hb, as_ref[h].reshape(NHID, 1),
                             preferred_element_type=jnp.float32))
        dcols.append(jnp.dot(hb, ad_ref[h].reshape(NHID, 1),
                             preferred_element_type=jnp.float32))
    sd_ref[...] = jnp.concatenate(scols + dcols, axis=1).T


def _make_tc1(interpret=False):
    def body(x_ref, w_ref, as_ref, ad_ref, ht0, ht1, ht2, ht3, sd_ref):
        _heads_matmul(x_ref[...], w_ref, as_ref, ad_ref, NHEADS,
                      (ht0, ht1, ht2, ht3), sd_ref)

    full = lambda shape: pl.BlockSpec(shape, lambda i: (0,) * len(shape))
    return pl.pallas_call(
        body,
        grid=(GRID,),
        in_specs=[
            pl.BlockSpec((BN, NFEAT), lambda i: (i, 0)),
            full((NHEADS, NFEAT, NHID)),
            full((NHEADS, NHID)),
            full((NHEADS, NHID)),
        ],
        out_specs=[pl.BlockSpec((BN, NHID), lambda i: (i, 0))] * NHEADS
        + [pl.BlockSpec((2 * NHEADS, BN), lambda i: (0, i))],
        out_shape=[jax.ShapeDtypeStruct((NP, NHID), jnp.float32)] * NHEADS
        + [jax.ShapeDtypeStruct((2 * NHEADS, NP), jnp.float32)],
        interpret=interpret,
    )


def _make_tc_mid(double_elu, interpret=False):
    def body(a0, a1, a2, a3, w_ref, as_ref, ad_ref, ht0, ht1, ht2, ht3, sd_ref):
        hcat = _norm_cols((a0, a1, a2, a3), NHEADS, double_elu)
        _heads_matmul(hcat, w_ref, as_ref, ad_ref, NHEADS,
                      (ht0, ht1, ht2, ht3), sd_ref)

    full = lambda shape: pl.BlockSpec(shape, lambda i: (0,) * len(shape))
    return pl.pallas_call(
        body,
        grid=(GRID,),
        in_specs=[pl.BlockSpec((NC, BN, ACOL), lambda i: (0, i, 0))] * NHEADS
        + [
            full((NHEADS, NFEAT, NHID)),
            full((NHEADS, NHID)),
            full((NHEADS, NHID)),
        ],
        out_specs=[pl.BlockSpec((BN, NHID), lambda i: (i, 0))] * NHEADS
        + [pl.BlockSpec((2 * NHEADS, BN), lambda i: (0, i))],
        out_shape=[jax.ShapeDtypeStruct((NP, NHID), jnp.float32)] * NHEADS
        + [jax.ShapeDtypeStruct((2 * NHEADS, NP), jnp.float32)],
        interpret=interpret,
    )


def _make_tc3(interpret=False):
    def body(a0, a1, a2, a3, w_ref, as_ref, ad_ref, ht_ref, sd_ref):
        hcat = _norm_cols((a0, a1, a2, a3), NHEADS, False)
        hb = jnp.dot(hcat, w_ref[...], preferred_element_type=jnp.float32)
        ht_ref[...] = hb
        s = jnp.dot(hb, as_ref[...].reshape(NHID, 1),
                    preferred_element_type=jnp.float32)
        d = jnp.dot(hb, ad_ref[...].reshape(NHID, 1),
                    preferred_element_type=jnp.float32)
        sd_ref[...] = jnp.concatenate([s, d], axis=1).T

    full = lambda shape: pl.BlockSpec(shape, lambda i: (0,) * len(shape))
    return pl.pallas_call(
        body,
        grid=(GRID,),
        in_specs=[pl.BlockSpec((NC, BN, ACOL), lambda i: (0, i, 0))] * NHEADS
        + [full((NFEAT, NHID)), full((NHID,)), full((NHID,))],
        out_specs=[
            pl.BlockSpec((BN, NHID), lambda i: (i, 0)),
            pl.BlockSpec((2, BN), lambda i: (0, i)),
        ],
        out_shape=[
            jax.ShapeDtypeStruct((NP, NHID), jnp.float32),
            jax.ShapeDtypeStruct((2, NP), jnp.float32),
        ],
        interpret=interpret,
    )


def _make_tc4(interpret=False):
    def body(a_ref, o_ref):
        pa = a_ref[0] + a_ref[1]
        o = pa[:, :NHID] / (pa[:, NHID:NHID + 1] + 1e-16)
        o_ref[...] = _elu(o)

    return pl.pallas_call(
        body,
        grid=(GRID,),
        in_specs=[pl.BlockSpec((NC, BN, ACOL), lambda i: (0, i, 0))],
        out_specs=pl.BlockSpec((BN, NHID), lambda i: (i, 0)),
        out_shape=jax.ShapeDtypeStruct((NP, NHID), jnp.float32),
        interpret=interpret,
    )


def _make_sc(nheads, interpret=False):
    mesh = plsc.VectorSubcoreMesh(core_axis_name="c", subcore_axis_name="s",
                                  num_cores=NC, num_subcores=NS)
    out_type = [jax.ShapeDtypeStruct((NC, NP, ACOL), jnp.float32)
                for _ in range(nheads)]
    scratch = [
        pltpu.VMEM((NCHUNK, K), jnp.int32),
        pltpu.VMEM((NCHUNK, K), jnp.int32),
        pltpu.VMEM((NP,), jnp.float32),
        pltpu.VMEM((NP,), jnp.float32),
        pltpu.VMEM((K,), jnp.float32),
        pltpu.VMEM((K, NHID), jnp.float32),
        pltpu.VMEM((K, ACOL), jnp.float32),
        pltpu.VMEM((ZR, ACOL), jnp.float32),
        pltpu.VMEM_SHARED((NP, ACOL), jnp.float32),
        pltpu.SemaphoreType.DMA,
    ]

    @functools.partial(
        pl.kernel, out_type=out_type, mesh=mesh, scratch_types=scratch,
        compiler_params=pltpu.CompilerParams(needs_layout_passes=False,
                                             use_tc_tiling_on_sc=False),
        interpret=interpret)
    def sck(src_hbm, dst_hbm, sd_hbm, *rest):
        ht = rest[:nheads]
        outs = rest[nheads:2 * nheads]
        (src_v, dst_v, s_v, d_v, ex_v, rows_v, st_v, zeros_v, acc_sh,
         sem) = rest[2 * nheads:]
        cid = lax.axis_index("c")
        sid = lax.axis_index("s")
        wid = cid * NS + sid
        pltpu.sync_copy(src_hbm.at[wid], src_v)
        pltpu.sync_copy(dst_hbm.at[wid], dst_v)
        z16 = jnp.zeros((16,), jnp.float32)
        for r in range(ZR):
            for q in range(ACOL // 16):
                zeros_v[r, pl.ds(q * 16, 16)] = z16
        iota16 = lax.iota(jnp.int32, 16)
        lanes = [jnp.full((16,), l, jnp.int32) for l in range(16)]
        row0 = sid * ROWS_PER_TILE

        for h in range(nheads):
            pltpu.sync_copy(sd_hbm.at[h], s_v)
            pltpu.sync_copy(sd_hbm.at[nheads + h], d_v)

            def zbody(i, _):
                pltpu.sync_copy(zeros_v, acc_sh.at[pl.ds(row0 + i * ZR, ZR)])
                return 0
            lax.fori_loop(0, ROWS_PER_TILE // ZR, zbody, 0)
            plsc.subcore_barrier()

            def chunk_body(c, _):
                for j in range(K // 16):
                    si = src_v[c, pl.ds(j * 16, 16)]
                    di = dst_v[c, pl.ds(j * 16, 16)]
                    sv = plsc.load_gather(s_v, [si])
                    dv = plsc.load_gather(d_v, [di])
                    e = sv + dv
                    e = jnp.where(e >= 0.0, e, e * NEG_SLOPE)
                    ex_v[pl.ds(j * 16, 16)] = jnp.exp(e)
                pltpu.async_copy(ht[h].at[src_v.at[c]], rows_v, sem).wait()

                def sbody(g, _):
                    ex16 = ex_v[pl.ds(g * 16, 16)]
                    base = g * 16
                    for l in range(16):
                        av = ex16.at[lanes[l]].get(mode="promise_in_bounds")
                        j = base + l
                        for q in range(NHID // 16):
                            st_v[j, pl.ds(q * 16, 16)] = (
                                rows_v[j, pl.ds(q * 16, 16)] * av)
                        st_v[j, pl.ds(NHID, 16)] = jnp.where(
                            iota16 == 0, av, 0.0)
                    return 0
                lax.fori_loop(0, K // 16, sbody, 0)
                pltpu.sync_copy(st_v, acc_sh.at[dst_v.at[c]], add=True)
                return 0
            lax.fori_loop(0, NCHUNK, chunk_body, 0)
            plsc.subcore_barrier()
            pltpu.sync_copy(
                acc_sh.at[pl.ds(row0, ROWS_PER_TILE)],
                outs[h].at[cid, pl.ds(row0, ROWS_PER_TILE)])
            plsc.subcore_barrier()

    return sck


_tc1 = _make_tc1()
_tc2 = _make_tc_mid(double_elu=True)
_tc3 = _make_tc3()
_tc4 = _make_tc4()

_SC_CACHE = {}


def _get_sc(nheads):
    if nheads not in _SC_CACHE:
        _SC_CACHE[nheads] = _make_sc(nheads)
    return _SC_CACHE[nheads]


@jax.jit
def _impl(x, edge_index, W1, a1_src, a1_dst, W2, a2_src, a2_dst,
          W3, a3_src, a3_dst):
    f32 = jnp.float32
    xp = jnp.zeros((NP, NFEAT), f32).at[:N].set(x.astype(f32))
    src = edge_index[0].astype(jnp.int32)
    dst = edge_index[1].astype(jnp.int32)
    pad = jnp.full((EPAD - E,), NP - 1, jnp.int32)
    src2 = jnp.concatenate([src, pad]).reshape(NW, NCHUNK, K)
    dst2 = jnp.concatenate([dst, pad]).reshape(NW, NCHUNK, K)
    W3p = jnp.zeros((NFEAT, NHID), f32).at[:, :NCLASS].set(W3)
    a3sp = jnp.zeros((NHID,), f32).at[:NCLASS].set(a3_src)
    a3dp = jnp.zeros((NHID,), f32).at[:NCLASS].set(a3_dst)

    sc4 = _get_sc(NHEADS)
    sc1 = _get_sc(1)
    *ht1, sd1 = _tc1(xp, W1, a1_src, a1_dst)
    acc1 = sc4(src2, dst2, sd1, *ht1)
    *ht2, sd2 = _tc2(*acc1, W2, a2_src, a2_dst)
    acc2 = sc4(src2, dst2, sd2, *ht2)
    ht3, sd3 = _tc3(*acc2, W3p, a3sp, a3dp)
    acc3 = sc1(src2, dst2, sd3, ht3)
    out = _tc4(acc3[0])
    return out[:N, :NCLASS]


def kernel(x, edge_index, n_node_features, mini_batch,
           W1, a1_src, a1_dst, W2, a2_src, a2_dst, W3, a3_src, a3_dst):
    return _impl(x, edge_index, W1, a1_src, a1_dst,
                 W2, a2_src, a2_dst, W3, a3_src, a3_dst)


# speedup vs baseline: 24.7816x; 1.5587x over previous
import functools

import jax
import jax.numpy as jnp
from jax import lax
from jax.experimental import pallas as pl
from jax.experimental.pallas import tpu as pltpu
from jax.experimental.pallas import tpu_sc as plsc

N = 10000
E = 160000
NFEAT = 256
NHID = 64
NHEADS = 4
NCLASS = 40
NEG_SLOPE = 0.2

NP = 10240
NC = 2
NS = 16
NW = NC * NS
K = 128
NCHUNK = 40
EPW = K * NCHUNK
EPAD = NW * EPW
ACOL = 80
ROWS_PER_TILE = NP // NS
ZR = 64

BN = 256
GRID = NP // BN


def _elu(v):
    return jnp.where(v > 0.0, v, jnp.exp(jnp.minimum(v, 0.0)) - 1.0)


def _norm_cols(acc_refs, nheads, double_elu):
    cols = []
    for h in range(nheads):
        pa = acc_refs[h][0] + acc_refs[h][1]
        o = pa[:, :NHID] / (pa[:, NHID:NHID + 1] + 1e-16)
        o = _elu(o)
        if double_elu:
            o = _elu(o)
        cols.append(o)
    return jnp.concatenate(cols, axis=1) if len(cols) > 1 else cols[0]


def _heads_matmul(xb, w_ref, as_ref, ad_ref, nheads, ht_refs, sd_ref):
    scols = []
    dcols = []
    for h in range(nheads):
        hb = jnp.dot(xb, w_ref[h], preferred_element_type=jnp.float32)
        ht_refs[h][...] = hb
        scols.append(jnp.dot(hb, as_ref[h].reshape(NHID, 1),
                             preferred_element_type=jnp.float32))
        dcols.append(jnp.dot(hb, ad_ref[h].reshape(NHID, 1),
                             preferred_element_type=jnp.float32))
    sd_ref[...] = jnp.concatenate(scols + dcols, axis=1).T


def _make_tc1(interpret=False):
    def body(x_ref, w_ref, as_ref, ad_ref, ht0, ht1, ht2, ht3, sd_ref):
        _heads_matmul(x_ref[...], w_ref, as_ref, ad_ref, NHEADS,
                      (ht0, ht1, ht2, ht3), sd_ref)

    full = lambda shape: pl.BlockSpec(shape, lambda i: (0,) * len(shape))
    return pl.pallas_call(
        body,
        grid=(GRID,),
        in_specs=[
            pl.BlockSpec((BN, NFEAT), lambda i: (i, 0)),
            full((NHEADS, NFEAT, NHID)),
            full((NHEADS, NHID)),
            full((NHEADS, NHID)),
        ],
        out_specs=[pl.BlockSpec((BN, NHID), lambda i: (i, 0))] * NHEADS
        + [pl.BlockSpec((2 * NHEADS, BN), lambda i: (0, i))],
        out_shape=[jax.ShapeDtypeStruct((NP, NHID), jnp.float32)] * NHEADS
        + [jax.ShapeDtypeStruct((2 * NHEADS, NP), jnp.float32)],
        interpret=interpret,
    )


def _make_tc_mid(double_elu, interpret=False):
    def body(a0, a1, a2, a3, w_ref, as_ref, ad_ref, ht0, ht1, ht2, ht3, sd_ref):
        hcat = _norm_cols((a0, a1, a2, a3), NHEADS, double_elu)
        _heads_matmul(hcat, w_ref, as_ref, ad_ref, NHEADS,
                      (ht0, ht1, ht2, ht3), sd_ref)

    full = lambda shape: pl.BlockSpec(shape, lambda i: (0,) * len(shape))
    return pl.pallas_call(
        body,
        grid=(GRID,),
        in_specs=[pl.BlockSpec((NC, BN, ACOL), lambda i: (0, i, 0))] * NHEADS
        + [
            full((NHEADS, NFEAT, NHID)),
            full((NHEADS, NHID)),
            full((NHEADS, NHID)),
        ],
        out_specs=[pl.BlockSpec((BN, NHID), lambda i: (i, 0))] * NHEADS
        + [pl.BlockSpec((2 * NHEADS, BN), lambda i: (0, i))],
        out_shape=[jax.ShapeDtypeStruct((NP, NHID), jnp.float32)] * NHEADS
        + [jax.ShapeDtypeStruct((2 * NHEADS, NP), jnp.float32)],
        interpret=interpret,
    )


def _make_tc3(interpret=False):
    def body(a0, a1, a2, a3, w_ref, as_ref, ad_ref, ht_ref, sd_ref):
        hcat = _norm_cols((a0, a1, a2, a3), NHEADS, False)
        hb = jnp.dot(hcat, w_ref[...], preferred_element_type=jnp.float32)
        ht_ref[...] = hb
        s = jnp.dot(hb, as_ref[...].reshape(NHID, 1),
                    preferred_element_type=jnp.float32)
        d = jnp.dot(hb, ad_ref[...].reshape(NHID, 1),
                    preferred_element_type=jnp.float32)
        sd_ref[...] = jnp.concatenate([s, d], axis=1).T

    full = lambda shape: pl.BlockSpec(shape, lambda i: (0,) * len(shape))
    return pl.pallas_call(
        body,
        grid=(GRID,),
        in_specs=[pl.BlockSpec((NC, BN, ACOL), lambda i: (0, i, 0))] * NHEADS
        + [full((NFEAT, NHID)), full((NHID,)), full((NHID,))],
        out_specs=[
            pl.BlockSpec((BN, NHID), lambda i: (i, 0)),
            pl.BlockSpec((2, BN), lambda i: (0, i)),
        ],
        out_shape=[
            jax.ShapeDtypeStruct((NP, NHID), jnp.float32),
            jax.ShapeDtypeStruct((2, NP), jnp.float32),
        ],
        interpret=interpret,
    )


def _make_tc4(interpret=False):
    def body(a_ref, o_ref):
        pa = a_ref[0] + a_ref[1]
        o = pa[:, :NHID] / (pa[:, NHID:NHID + 1] + 1e-16)
        o_ref[...] = _elu(o)

    return pl.pallas_call(
        body,
        grid=(GRID,),
        in_specs=[pl.BlockSpec((NC, BN, ACOL), lambda i: (0, i, 0))],
        out_specs=pl.BlockSpec((BN, NHID), lambda i: (i, 0)),
        out_shape=jax.ShapeDtypeStruct((NP, NHID), jnp.float32),
        interpret=interpret,
    )


def _make_sc(nheads, interpret=False):
    mesh = plsc.VectorSubcoreMesh(core_axis_name="c", subcore_axis_name="s",
                                  num_cores=NC, num_subcores=NS)
    out_type = [jax.ShapeDtypeStruct((NC, NP, ACOL), jnp.float32)
                for _ in range(nheads)]
    scratch = [
        pltpu.VMEM((NCHUNK, K), jnp.int32),
        pltpu.VMEM((NCHUNK, K), jnp.int32),
        pltpu.VMEM((NP,), jnp.float32),
        pltpu.VMEM((NP,), jnp.float32),
        pltpu.VMEM((K,), jnp.float32),
        pltpu.VMEM((K, NHID), jnp.float32),
        pltpu.VMEM((K, NHID), jnp.float32),
        pltpu.VMEM((K, ACOL), jnp.float32),
        pltpu.VMEM((K, ACOL), jnp.float32),
        pltpu.VMEM((ZR, ACOL), jnp.float32),
        pltpu.VMEM_SHARED((NP, ACOL), jnp.float32),
        pltpu.SemaphoreType.DMA,
        pltpu.SemaphoreType.DMA,
        pltpu.SemaphoreType.DMA,
    ]

    @functools.partial(
        pl.kernel, out_type=out_type, mesh=mesh, scratch_types=scratch,
        compiler_params=pltpu.CompilerParams(needs_layout_passes=False,
                                             use_tc_tiling_on_sc=False),
        interpret=interpret)
    def sck(src_hbm, dst_hbm, sd_hbm, *rest):
        ht = rest[:nheads]
        outs = rest[nheads:2 * nheads]
        (src_v, dst_v, s_v, d_v, ex_v, rows_v0, rows_v1, st_v0, st_v1,
         zeros_v, acc_sh, gsem0, gsem1, zsem) = rest[2 * nheads:]
        cid = lax.axis_index("c")
        sid = lax.axis_index("s")
        wid = cid * NS + sid
        pltpu.sync_copy(src_hbm.at[wid], src_v)
        pltpu.sync_copy(dst_hbm.at[wid], dst_v)
        z16 = jnp.zeros((16,), jnp.float32)
        for r in range(ZR):
            for q in range(ACOL // 16):
                zeros_v[r, pl.ds(q * 16, 16)] = z16
        iota16 = lax.iota(jnp.int32, 16)
        lanes = [jnp.full((16,), l, jnp.int32) for l in range(16)]
        row0 = sid * ROWS_PER_TILE

        def compute_ex(c):
            for j in range(K // 16):
                si = src_v[c, pl.ds(j * 16, 16)]
                di = dst_v[c, pl.ds(j * 16, 16)]
                sv = plsc.load_gather(s_v, [si])
                dv = plsc.load_gather(d_v, [di])
                e = sv + dv
                e = jnp.where(e >= 0.0, e, e * NEG_SLOPE)
                ex_v[pl.ds(j * 16, 16)] = jnp.exp(e)

        def scale(rows_v, st_v):
            def sbody(g, _):
                ex16 = ex_v[pl.ds(g * 16, 16)]
                base = g * 16
                for l in range(16):
                    av = ex16.at[lanes[l]].get(mode="promise_in_bounds")
                    j = base + l
                    for q in range(NHID // 16):
                        st_v[j, pl.ds(q * 16, 16)] = (
                            rows_v[j, pl.ds(q * 16, 16)] * av)
                    st_v[j, pl.ds(NHID, 16)] = jnp.where(
                        iota16 == 0, av, 0.0)
                return 0
            lax.fori_loop(0, K // 16, sbody, 0)

        for h in range(nheads):
            pltpu.sync_copy(sd_hbm.at[h], s_v)
            pltpu.sync_copy(sd_hbm.at[nheads + h], d_v)

            zd = [pltpu.async_copy(
                zeros_v, acc_sh.at[pl.ds(row0 + b * ZR, ZR)], zsem)
                for b in range(ROWS_PER_TILE // ZR)]
            for dsc in zd:
                dsc.wait()
            plsc.subcore_barrier()

            g0 = pltpu.async_copy(ht[h].at[src_v.at[0]], rows_v0, gsem0)

            def pair_body(i, _):
                c = 2 * i
                g1 = pltpu.async_copy(ht[h].at[src_v.at[c + 1]], rows_v1,
                                      gsem1)
                pltpu.make_async_copy(ht[h].at[src_v.at[c]], rows_v0,
                                      gsem0).wait()
                compute_ex(c)
                scale(rows_v0, st_v0)
                pltpu.sync_copy(st_v0, acc_sh.at[dst_v.at[c]], add=True)

                @pl.when(c + 2 < NCHUNK)
                def _():
                    pltpu.async_copy(ht[h].at[src_v.at[c + 2]], rows_v0,
                                     gsem0)
                g1.wait()
                compute_ex(c + 1)
                scale(rows_v1, st_v1)
                pltpu.sync_copy(st_v1, acc_sh.at[dst_v.at[c + 1]], add=True)
                return 0
            lax.fori_loop(0, NCHUNK // 2, pair_body, 0)
            plsc.subcore_barrier()
            pltpu.sync_copy(
                acc_sh.at[pl.ds(row0, ROWS_PER_TILE)],
                outs[h].at[cid, pl.ds(row0, ROWS_PER_TILE)])
            plsc.subcore_barrier()

    return sck


_tc1 = _make_tc1()
_tc2 = _make_tc_mid(double_elu=True)
_tc3 = _make_tc3()
_tc4 = _make_tc4()

_SC_CACHE = {}


def _get_sc(nheads):
    if nheads not in _SC_CACHE:
        _SC_CACHE[nheads] = _make_sc(nheads)
    return _SC_CACHE[nheads]


@jax.jit
def _impl(x, edge_index, W1, a1_src, a1_dst, W2, a2_src, a2_dst,
          W3, a3_src, a3_dst):
    f32 = jnp.float32
    xp = jnp.zeros((NP, NFEAT), f32).at[:N].set(x.astype(f32))
    src = edge_index[0].astype(jnp.int32)
    dst = edge_index[1].astype(jnp.int32)
    pad = jnp.full((EPAD - E,), NP - 1, jnp.int32)
    src2 = jnp.concatenate([src, pad]).reshape(NW, NCHUNK, K)
    dst2 = jnp.concatenate([dst, pad]).reshape(NW, NCHUNK, K)
    W3p = jnp.zeros((NFEAT, NHID), f32).at[:, :NCLASS].set(W3)
    a3sp = jnp.zeros((NHID,), f32).at[:NCLASS].set(a3_src)
    a3dp = jnp.zeros((NHID,), f32).at[:NCLASS].set(a3_dst)

    sc4 = _get_sc(NHEADS)
    sc1 = _get_sc(1)
    *ht1, sd1 = _tc1(xp, W1, a1_src, a1_dst)
    acc1 = sc4(src2, dst2, sd1, *ht1)
    *ht2, sd2 = _tc2(*acc1, W2, a2_src, a2_dst)
    acc2 = sc4(src2, dst2, sd2, *ht2)
    ht3, sd3 = _tc3(*acc2, W3p, a3sp, a3dp)
    acc3 = sc1(src2, dst2, sd3, ht3)
    out = _tc4(acc3[0])
    return out[:N, :NCLASS]


def kernel(x, edge_index, n_node_features, mini_batch,
           W1, a1_src, a1_dst, W2, a2_src, a2_dst, W3, a3_src, a3_dst):
    return _impl(x, edge_index, W1, a1_src, a1_dst,
                 W2, a2_src, a2_dst, W3, a3_src, a3_dst)


# speedup vs baseline: 25.4593x; 1.0273x over previous
import functools

import jax
import jax.numpy as jnp
from jax import lax
from jax.experimental import pallas as pl
from jax.experimental.pallas import tpu as pltpu
from jax.experimental.pallas import tpu_sc as plsc

N = 10000
E = 160000
NFEAT = 256
NHID = 64
NHEADS = 4
NCLASS = 40
NEG_SLOPE = 0.2

NP = 10240
NC = 2
NS = 16
NW = NC * NS
K = 128
NCHUNK = 40
EPW = K * NCHUNK
EPAD = NW * EPW
ACOL = 80
ROWS_PER_TILE = NP // NS
ZR = 64

BN = 256
GRID = NP // BN


def _elu(v):
    return jnp.where(v > 0.0, v, jnp.exp(jnp.minimum(v, 0.0)) - 1.0)


def _norm_cols(acc_refs, nheads, double_elu):
    cols = []
    for h in range(nheads):
        pa = acc_refs[h][0] + acc_refs[h][1]
        o = pa[:, :NHID] / (pa[:, NHID:NHID + 1] + 1e-16)
        o = _elu(o)
        if double_elu:
            o = _elu(o)
        cols.append(o)
    return jnp.concatenate(cols, axis=1) if len(cols) > 1 else cols[0]


def _heads_matmul(xb, w_ref, as_ref, ad_ref, nheads, ht_refs, sd_ref):
    scols = []
    dcols = []
    for h in range(nheads):
        hb = jnp.dot(xb, w_ref[h], preferred_element_type=jnp.float32)
        ht_refs[h][...] = hb
        scols.append(jnp.dot(hb, as_ref[h].reshape(NHID, 1),
                             preferred_element_type=jnp.float32))
        dcols.append(jnp.dot(hb, ad_ref[h].reshape(NHID, 1),
                             preferred_element_type=jnp.float32))
    sd_ref[...] = jnp.concatenate(scols + dcols, axis=1).T


def _make_tc1(interpret=False):
    def body(x_ref, w_ref, as_ref, ad_ref, ht0, ht1, ht2, ht3, sd_ref):
        _heads_matmul(x_ref[...], w_ref, as_ref, ad_ref, NHEADS,
                      (ht0, ht1, ht2, ht3), sd_ref)

    full = lambda shape: pl.BlockSpec(shape, lambda i: (0,) * len(shape))
    return pl.pallas_call(
        body,
        grid=(GRID,),
        in_specs=[
            pl.BlockSpec((BN, NFEAT), lambda i: (i, 0)),
            full((NHEADS, NFEAT, NHID)),
            full((NHEADS, NHID)),
            full((NHEADS, NHID)),
        ],
        out_specs=[pl.BlockSpec((BN, NHID), lambda i: (i, 0))] * NHEADS
        + [pl.BlockSpec((2 * NHEADS, BN), lambda i: (0, i))],
        out_shape=[jax.ShapeDtypeStruct((NP, NHID), jnp.float32)] * NHEADS
        + [jax.ShapeDtypeStruct((2 * NHEADS, NP), jnp.float32)],
        interpret=interpret,
    )


def _make_tc_mid(double_elu, interpret=False):
    def body(a0, a1, a2, a3, w_ref, as_ref, ad_ref, ht0, ht1, ht2, ht3, sd_ref):
        hcat = _norm_cols((a0, a1, a2, a3), NHEADS, double_elu)
        _heads_matmul(hcat, w_ref, as_ref, ad_ref, NHEADS,
                      (ht0, ht1, ht2, ht3), sd_ref)

    full = lambda shape: pl.BlockSpec(shape, lambda i: (0,) * len(shape))
    return pl.pallas_call(
        body,
        grid=(GRID,),
        in_specs=[pl.BlockSpec((NC, BN, ACOL), lambda i: (0, i, 0))] * NHEADS
        + [
            full((NHEADS, NFEAT, NHID)),
            full((NHEADS, NHID)),
            full((NHEADS, NHID)),
        ],
        out_specs=[pl.BlockSpec((BN, NHID), lambda i: (i, 0))] * NHEADS
        + [pl.BlockSpec((2 * NHEADS, BN), lambda i: (0, i))],
        out_shape=[jax.ShapeDtypeStruct((NP, NHID), jnp.float32)] * NHEADS
        + [jax.ShapeDtypeStruct((2 * NHEADS, NP), jnp.float32)],
        interpret=interpret,
    )


def _make_tc3(interpret=False):
    def body(a0, a1, a2, a3, w_ref, as_ref, ad_ref, ht_ref, sd_ref):
        hcat = _norm_cols((a0, a1, a2, a3), NHEADS, False)
        hb = jnp.dot(hcat, w_ref[...], preferred_element_type=jnp.float32)
        ht_ref[...] = hb
        s = jnp.dot(hb, as_ref[...].reshape(NHID, 1),
                    preferred_element_type=jnp.float32)
        d = jnp.dot(hb, ad_ref[...].reshape(NHID, 1),
                    preferred_element_type=jnp.float32)
        sd_ref[...] = jnp.concatenate([s, d], axis=1).T

    full = lambda shape: pl.BlockSpec(shape, lambda i: (0,) * len(shape))
    return pl.pallas_call(
        body,
        grid=(GRID,),
        in_specs=[pl.BlockSpec((NC, BN, ACOL), lambda i: (0, i, 0))] * NHEADS
        + [full((NFEAT, NHID)), full((NHID,)), full((NHID,))],
        out_specs=[
            pl.BlockSpec((BN, NHID), lambda i: (i, 0)),
            pl.BlockSpec((2, BN), lambda i: (0, i)),
        ],
        out_shape=[
            jax.ShapeDtypeStruct((NP, NHID), jnp.float32),
            jax.ShapeDtypeStruct((2, NP), jnp.float32),
        ],
        interpret=interpret,
    )


def _make_tc4(interpret=False):
    def body(a_ref, o_ref):
        pa = a_ref[0] + a_ref[1]
        o = pa[:, :NHID] / (pa[:, NHID:NHID + 1] + 1e-16)
        o_ref[...] = _elu(o)

    return pl.pallas_call(
        body,
        grid=(GRID,),
        in_specs=[pl.BlockSpec((NC, BN, ACOL), lambda i: (0, i, 0))],
        out_specs=pl.BlockSpec((BN, NHID), lambda i: (i, 0)),
        out_shape=jax.ShapeDtypeStruct((NP, NHID), jnp.float32),
        interpret=interpret,
    )


def _make_sc(nheads, interpret=False):
    mesh = plsc.VectorSubcoreMesh(core_axis_name="c", subcore_axis_name="s",
                                  num_cores=NC, num_subcores=NS)
    out_type = [jax.ShapeDtypeStruct((NC, NP, ACOL), jnp.float32)
                for _ in range(nheads)]
    scratch = [
        pltpu.VMEM((NCHUNK, K), jnp.int32),
        pltpu.VMEM((NCHUNK, K), jnp.int32),
        pltpu.VMEM((NP,), jnp.float32),
        pltpu.VMEM((NP,), jnp.float32),
        pltpu.VMEM((K,), jnp.float32),
        pltpu.VMEM((K, NHID), jnp.float32),
        pltpu.VMEM((K, NHID), jnp.float32),
        pltpu.VMEM((K, ACOL), jnp.float32),
        pltpu.VMEM((K, ACOL), jnp.float32),
        pltpu.VMEM((ZR, ACOL), jnp.float32),
        pltpu.VMEM_SHARED((NP, ACOL), jnp.float32),
        pltpu.SemaphoreType.DMA,
        pltpu.SemaphoreType.DMA,
        pltpu.SemaphoreType.DMA,
        pltpu.SemaphoreType.DMA,
        pltpu.SemaphoreType.DMA,
    ]

    @functools.partial(
        pl.kernel, out_type=out_type, mesh=mesh, scratch_types=scratch,
        compiler_params=pltpu.CompilerParams(needs_layout_passes=False,
                                             use_tc_tiling_on_sc=False),
        interpret=interpret)
    def sck(src_hbm, dst_hbm, sd_hbm, *rest):
        ht = rest[:nheads]
        outs = rest[nheads:2 * nheads]
        (src_v, dst_v, s_v, d_v, ex_v, rows_v0, rows_v1, st_v0, st_v1,
         zeros_v, acc_sh, gsem0, gsem1, zsem, ssem0, ssem1) = rest[2 * nheads:]
        cid = lax.axis_index("c")
        sid = lax.axis_index("s")
        wid = cid * NS + sid
        pltpu.sync_copy(src_hbm.at[wid], src_v)
        pltpu.sync_copy(dst_hbm.at[wid], dst_v)
        z16 = jnp.zeros((16,), jnp.float32)
        for r in range(ZR):
            for q in range(ACOL // 16):
                zeros_v[r, pl.ds(q * 16, 16)] = z16
        iota16 = lax.iota(jnp.int32, 16)
        lanes = [jnp.full((16,), l, jnp.int32) for l in range(16)]
        row0 = sid * ROWS_PER_TILE

        def compute_ex(c):
            for j in range(K // 16):
                si = src_v[c, pl.ds(j * 16, 16)]
                di = dst_v[c, pl.ds(j * 16, 16)]
                sv = plsc.load_gather(s_v, [si])
                dv = plsc.load_gather(d_v, [di])
                e = sv + dv
                e = jnp.where(e >= 0.0, e, e * NEG_SLOPE)
                ex_v[pl.ds(j * 16, 16)] = jnp.exp(e)

        def scale(rows_v, st_v):
            def sbody(g, _):
                ex16 = ex_v[pl.ds(g * 16, 16)]
                base = g * 16
                for l in range(16):
                    av = ex16.at[lanes[l]].get(mode="promise_in_bounds")
                    j = base + l
                    for q in range(NHID // 16):
                        st_v[j, pl.ds(q * 16, 16)] = (
                            rows_v[j, pl.ds(q * 16, 16)] * av)
                    st_v[j, pl.ds(NHID, 16)] = jnp.where(
                        iota16 == 0, av, 0.0)
                return 0
            lax.fori_loop(0, K // 16, sbody, 0)

        for h in range(nheads):
            pltpu.sync_copy(sd_hbm.at[h], s_v)
            pltpu.sync_copy(sd_hbm.at[nheads + h], d_v)

            zd = [pltpu.async_copy(
                zeros_v, acc_sh.at[pl.ds(row0 + b * ZR, ZR)], zsem)
                for b in range(ROWS_PER_TILE // ZR)]
            for dsc in zd:
                dsc.wait()
            plsc.subcore_barrier()

            g0 = pltpu.async_copy(ht[h].at[src_v.at[0]], rows_v0, gsem0)

            def pair_body(i, _):
                c = 2 * i
                g1 = pltpu.async_copy(ht[h].at[src_v.at[c + 1]], rows_v1,
                                      gsem1)
                pltpu.make_async_copy(ht[h].at[src_v.at[c]], rows_v0,
                                      gsem0).wait()
                compute_ex(c)

                @pl.when(c >= 2)
                def _():
                    pltpu.make_async_copy(st_v0, acc_sh.at[dst_v.at[c]],
                                          ssem0).wait()
                scale(rows_v0, st_v0)
                pltpu.async_copy(st_v0, acc_sh.at[dst_v.at[c]], ssem0,
                                 add=True)

                @pl.when(c + 2 < NCHUNK)
                def _():
                    pltpu.async_copy(ht[h].at[src_v.at[c + 2]], rows_v0,
                                     gsem0)
                g1.wait()
                compute_ex(c + 1)

                @pl.when(c >= 2)
                def _():
                    pltpu.make_async_copy(st_v1, acc_sh.at[dst_v.at[c + 1]],
                                          ssem1).wait()
                scale(rows_v1, st_v1)
                pltpu.async_copy(st_v1, acc_sh.at[dst_v.at[c + 1]], ssem1,
                                 add=True)
                return 0
            lax.fori_loop(0, NCHUNK // 2, pair_body, 0)
            pltpu.make_async_copy(st_v0, acc_sh.at[dst_v.at[NCHUNK - 2]],
                                  ssem0).wait()
            pltpu.make_async_copy(st_v1, acc_sh.at[dst_v.at[NCHUNK - 1]],
                                  ssem1).wait()
            plsc.subcore_barrier()
            pltpu.sync_copy(
                acc_sh.at[pl.ds(row0, ROWS_PER_TILE)],
                outs[h].at[cid, pl.ds(row0, ROWS_PER_TILE)])
            plsc.subcore_barrier()

    return sck


_tc1 = _make_tc1()
_tc2 = _make_tc_mid(double_elu=True)
_tc3 = _make_tc3()
_tc4 = _make_tc4()

_SC_CACHE = {}


def _get_sc(nheads):
    if nheads not in _SC_CACHE:
        _SC_CACHE[nheads] = _make_sc(nheads)
    return _SC_CACHE[nheads]


@jax.jit
def _impl(x, edge_index, W1, a1_src, a1_dst, W2, a2_src, a2_dst,
          W3, a3_src, a3_dst):
    f32 = jnp.float32
    xp = jnp.zeros((NP, NFEAT), f32).at[:N].set(x.astype(f32))
    src = edge_index[0].astype(jnp.int32)
    dst = edge_index[1].astype(jnp.int32)
    pad = jnp.full((EPAD - E,), NP - 1, jnp.int32)
    src2 = jnp.concatenate([src, pad]).reshape(NW, NCHUNK, K)
    dst2 = jnp.concatenate([dst, pad]).reshape(NW, NCHUNK, K)
    W3p = jnp.zeros((NFEAT, NHID), f32).at[:, :NCLASS].set(W3)
    a3sp = jnp.zeros((NHID,), f32).at[:NCLASS].set(a3_src)
    a3dp = jnp.zeros((NHID,), f32).at[:NCLASS].set(a3_dst)

    sc4 = _get_sc(NHEADS)
    sc1 = _get_sc(1)
    *ht1, sd1 = _tc1(xp, W1, a1_src, a1_dst)
    acc1 = sc4(src2, dst2, sd1, *ht1)
    *ht2, sd2 = _tc2(*acc1, W2, a2_src, a2_dst)
    acc2 = sc4(src2, dst2, sd2, *ht2)
    ht3, sd3 = _tc3(*acc2, W3p, a3sp, a3dp)
    acc3 = sc1(src2, dst2, sd3, ht3)
    out = _tc4(acc3[0])
    return out[:N, :NCLASS]


def kernel(x, edge_index, n_node_features, mini_batch,
           W1, a1_src, a1_dst, W2, a2_src, a2_dst, W3, a3_src, a3_dst):
    return _impl(x, edge_index, W1, a1_src, a1_dst,
                 W2, a2_src, a2_dst, W3, a3_src, a3_dst)


# speedup vs baseline: 27.6782x; 1.0872x over previous
import functools

import jax
import jax.numpy as jnp
from jax import lax
from jax.experimental import pallas as pl
from jax.experimental.pallas import tpu as pltpu
from jax.experimental.pallas import tpu_sc as plsc

N = 10000
E = 160000
NFEAT = 256
NHID = 64
NHEADS = 4
NCLASS = 40
NEG_SLOPE = 0.2

NP = 10240
NC = 2
NS = 16
NW = NC * NS
K = 128
NCHUNK = 40
EPW = K * NCHUNK
EPAD = NW * EPW
ACOL = 80
ROWS_PER_TILE = NP // NS
ZR = 64

BN = 256
GRID = NP // BN


def _elu(v):
    return jnp.where(v > 0.0, v, jnp.exp(jnp.minimum(v, 0.0)) - 1.0)


def _norm_cols(acc_refs, nheads, double_elu):
    cols = []
    for h in range(nheads):
        pa = acc_refs[h][0] + acc_refs[h][1]
        o = pa[:, :NHID] / (pa[:, NHID:NHID + 1] + 1e-16)
        o = _elu(o)
        if double_elu:
            o = _elu(o)
        cols.append(o)
    return jnp.concatenate(cols, axis=1) if len(cols) > 1 else cols[0]


def _heads_matmul(xb, w_ref, as_ref, ad_ref, nheads, ht_refs, sd_ref):
    scols = []
    dcols = []
    for h in range(nheads):
        hb = jnp.dot(xb, w_ref[h], preferred_element_type=jnp.float32)
        ht_refs[h][...] = hb
        scols.append(jnp.dot(hb, as_ref[h].reshape(NHID, 1),
                             preferred_element_type=jnp.float32))
        dcols.append(jnp.dot(hb, ad_ref[h].reshape(NHID, 1),
                             preferred_element_type=jnp.float32))
    sd_ref[...] = jnp.concatenate(scols + dcols, axis=1).T


def _make_tc1(interpret=False):
    def body(x_ref, w_ref, as_ref, ad_ref, ht0, ht1, ht2, ht3, sd_ref):
        _heads_matmul(x_ref[...], w_ref, as_ref, ad_ref, NHEADS,
                      (ht0, ht1, ht2, ht3), sd_ref)

    full = lambda shape: pl.BlockSpec(shape, lambda i: (0,) * len(shape))
    return pl.pallas_call(
        body,
        grid=(GRID,),
        in_specs=[
            pl.BlockSpec((BN, NFEAT), lambda i: (i, 0)),
            full((NHEADS, NFEAT, NHID)),
            full((NHEADS, NHID)),
            full((NHEADS, NHID)),
        ],
        out_specs=[pl.BlockSpec((BN, NHID), lambda i: (i, 0))] * NHEADS
        + [pl.BlockSpec((2 * NHEADS, BN), lambda i: (0, i))],
        out_shape=[jax.ShapeDtypeStruct((NP, NHID), jnp.float32)] * NHEADS
        + [jax.ShapeDtypeStruct((2 * NHEADS, NP), jnp.float32)],
        interpret=interpret,
    )


def _make_tc_mid(double_elu, interpret=False):
    def body(a0, a1, a2, a3, w_ref, as_ref, ad_ref, ht0, ht1, ht2, ht3, sd_ref):
        hcat = _norm_cols((a0, a1, a2, a3), NHEADS, double_elu)
        _heads_matmul(hcat, w_ref, as_ref, ad_ref, NHEADS,
                      (ht0, ht1, ht2, ht3), sd_ref)

    full = lambda shape: pl.BlockSpec(shape, lambda i: (0,) * len(shape))
    return pl.pallas_call(
        body,
        grid=(GRID,),
        in_specs=[pl.BlockSpec((NC, BN, ACOL), lambda i: (0, i, 0))] * NHEADS
        + [
            full((NHEADS, NFEAT, NHID)),
            full((NHEADS, NHID)),
            full((NHEADS, NHID)),
        ],
        out_specs=[pl.BlockSpec((BN, NHID), lambda i: (i, 0))] * NHEADS
        + [pl.BlockSpec((2 * NHEADS, BN), lambda i: (0, i))],
        out_shape=[jax.ShapeDtypeStruct((NP, NHID), jnp.float32)] * NHEADS
        + [jax.ShapeDtypeStruct((2 * NHEADS, NP), jnp.float32)],
        interpret=interpret,
    )


def _make_tc3(interpret=False):
    def body(a0, a1, a2, a3, w_ref, as_ref, ad_ref, ht_ref, sd_ref):
        hcat = _norm_cols((a0, a1, a2, a3), NHEADS, False)
        hb = jnp.dot(hcat, w_ref[...], preferred_element_type=jnp.float32)
        ht_ref[...] = hb
        s = jnp.dot(hb, as_ref[...].reshape(NHID, 1),
                    preferred_element_type=jnp.float32)
        d = jnp.dot(hb, ad_ref[...].reshape(NHID, 1),
                    preferred_element_type=jnp.float32)
        sd_ref[...] = jnp.concatenate([s, d], axis=1).T

    full = lambda shape: pl.BlockSpec(shape, lambda i: (0,) * len(shape))
    return pl.pallas_call(
        body,
        grid=(GRID,),
        in_specs=[pl.BlockSpec((NC, BN, ACOL), lambda i: (0, i, 0))] * NHEADS
        + [full((NFEAT, NHID)), full((NHID,)), full((NHID,))],
        out_specs=[
            pl.BlockSpec((BN, NHID), lambda i: (i, 0)),
            pl.BlockSpec((2, BN), lambda i: (0, i)),
        ],
        out_shape=[
            jax.ShapeDtypeStruct((NP, NHID), jnp.float32),
            jax.ShapeDtypeStruct((2, NP), jnp.float32),
        ],
        interpret=interpret,
    )


def _make_tc4(interpret=False):
    def body(a_ref, o_ref):
        pa = a_ref[0] + a_ref[1]
        o = pa[:, :NHID] / (pa[:, NHID:NHID + 1] + 1e-16)
        o_ref[...] = _elu(o)

    return pl.pallas_call(
        body,
        grid=(GRID,),
        in_specs=[pl.BlockSpec((NC, BN, ACOL), lambda i: (0, i, 0))],
        out_specs=pl.BlockSpec((BN, NHID), lambda i: (i, 0)),
        out_shape=jax.ShapeDtypeStruct((NP, NHID), jnp.float32),
        interpret=interpret,
    )


def _make_sc(nheads, interpret=False):
    mesh = plsc.VectorSubcoreMesh(core_axis_name="c", subcore_axis_name="s",
                                  num_cores=NC, num_subcores=NS)
    out_type = [jax.ShapeDtypeStruct((NC, NP, ACOL), jnp.float32)
                for _ in range(nheads)]
    scratch = [
        pltpu.VMEM((NCHUNK, K), jnp.int32),
        pltpu.VMEM((NCHUNK, K), jnp.int32),
        pltpu.VMEM((NP,), jnp.float32),
        pltpu.VMEM((NP,), jnp.float32),
        pltpu.VMEM((K,), jnp.float32),
        pltpu.VMEM((K, NHID), jnp.float32),
        pltpu.VMEM((K, NHID), jnp.float32),
        pltpu.VMEM((K, ACOL), jnp.float32),
        pltpu.VMEM((K, ACOL), jnp.float32),
        pltpu.VMEM((ZR, ACOL), jnp.float32),
        pltpu.VMEM_SHARED((NP, ACOL), jnp.float32),
        pltpu.SemaphoreType.DMA,
        pltpu.SemaphoreType.DMA,
        pltpu.SemaphoreType.DMA,
        pltpu.SemaphoreType.DMA,
        pltpu.SemaphoreType.DMA,
    ]

    @functools.partial(
        pl.kernel, out_type=out_type, mesh=mesh, scratch_types=scratch,
        compiler_params=pltpu.CompilerParams(needs_layout_passes=False,
                                             use_tc_tiling_on_sc=False),
        interpret=interpret)
    def sck(src_hbm, dst_hbm, sd_hbm, *rest):
        ht = rest[:nheads]
        outs = rest[nheads:2 * nheads]
        (src_v, dst_v, s_v, d_v, ex_v, rows_v0, rows_v1, st_v0, st_v1,
         zeros_v, acc_sh, gsem0, gsem1, zsem, ssem0, ssem1) = rest[2 * nheads:]
        cid = lax.axis_index("c")
        sid = lax.axis_index("s")
        wid = cid * NS + sid
        pltpu.sync_copy(src_hbm.at[wid], src_v)
        pltpu.sync_copy(dst_hbm.at[wid], dst_v)
        z16 = jnp.zeros((16,), jnp.float32)
        for r in range(ZR):
            for q in range(ACOL // 16):
                zeros_v[r, pl.ds(q * 16, 16)] = z16
        iota16 = lax.iota(jnp.int32, 16)
        lanes = [jnp.full((16,), l, jnp.int32) for l in range(16)]
        row0 = sid * ROWS_PER_TILE

        def compute_ex(c):
            @plsc.parallel_loop(0, K // 16)
            def _exbody(j):
                si = src_v[c, pl.ds(j * 16, 16)]
                di = dst_v[c, pl.ds(j * 16, 16)]
                sv = plsc.load_gather(s_v, [si])
                dv = plsc.load_gather(d_v, [di])
                e = sv + dv
                e = jnp.where(e >= 0.0, e, e * NEG_SLOPE)
                ex_v[pl.ds(j * 16, 16)] = jnp.exp(e)

        def scale(rows_v, st_v):
            @plsc.parallel_loop(0, K // 16)
            def sbody(g):
                ex16 = ex_v[pl.ds(g * 16, 16)]
                base = g * 16
                for l in range(16):
                    av = ex16.at[lanes[l]].get(mode="promise_in_bounds")
                    j = base + l
                    for q in range(NHID // 16):
                        st_v[j, pl.ds(q * 16, 16)] = (
                            rows_v[j, pl.ds(q * 16, 16)] * av)
                    st_v[j, pl.ds(NHID, 16)] = jnp.where(
                        iota16 == 0, av, 0.0)

        for h in range(nheads):
            pltpu.sync_copy(sd_hbm.at[h], s_v)
            pltpu.sync_copy(sd_hbm.at[nheads + h], d_v)

            zd = [pltpu.async_copy(
                zeros_v, acc_sh.at[pl.ds(row0 + b * ZR, ZR)], zsem)
                for b in range(ROWS_PER_TILE // ZR)]
            for dsc in zd:
                dsc.wait()
            plsc.subcore_barrier()

            g0 = pltpu.async_copy(ht[h].at[src_v.at[0]], rows_v0, gsem0)

            def pair_body(i, _):
                c = 2 * i
                g1 = pltpu.async_copy(ht[h].at[src_v.at[c + 1]], rows_v1,
                                      gsem1)
                pltpu.make_async_copy(ht[h].at[src_v.at[c]], rows_v0,
                                      gsem0).wait()
                compute_ex(c)

                @pl.when(c >= 2)
                def _():
                    pltpu.make_async_copy(st_v0, acc_sh.at[dst_v.at[c]],
                                          ssem0).wait()
                scale(rows_v0, st_v0)
                pltpu.async_copy(st_v0, acc_sh.at[dst_v.at[c]], ssem0,
                                 add=True)

                @pl.when(c + 2 < NCHUNK)
                def _():
                    pltpu.async_copy(ht[h].at[src_v.at[c + 2]], rows_v0,
                                     gsem0)
                g1.wait()
                compute_ex(c + 1)

                @pl.when(c >= 2)
                def _():
                    pltpu.make_async_copy(st_v1, acc_sh.at[dst_v.at[c + 1]],
                                          ssem1).wait()
                scale(rows_v1, st_v1)
                pltpu.async_copy(st_v1, acc_sh.at[dst_v.at[c + 1]], ssem1,
                                 add=True)
                return 0
            lax.fori_loop(0, NCHUNK // 2, pair_body, 0)
            pltpu.make_async_copy(st_v0, acc_sh.at[dst_v.at[NCHUNK - 2]],
                                  ssem0).wait()
            pltpu.make_async_copy(st_v1, acc_sh.at[dst_v.at[NCHUNK - 1]],
                                  ssem1).wait()
            plsc.subcore_barrier()
            pltpu.sync_copy(
                acc_sh.at[pl.ds(row0, ROWS_PER_TILE)],
                outs[h].at[cid, pl.ds(row0, ROWS_PER_TILE)])
            plsc.subcore_barrier()

    return sck


_tc1 = _make_tc1()
_tc2 = _make_tc_mid(double_elu=True)
_tc3 = _make_tc3()
_tc4 = _make_tc4()

_SC_CACHE = {}


def _get_sc(nheads):
    if nheads not in _SC_CACHE:
        _SC_CACHE[nheads] = _make_sc(nheads)
    return _SC_CACHE[nheads]


@jax.jit
def _impl(x, edge_index, W1, a1_src, a1_dst, W2, a2_src, a2_dst,
          W3, a3_src, a3_dst):
    f32 = jnp.float32
    xp = jnp.zeros((NP, NFEAT), f32).at[:N].set(x.astype(f32))
    src = edge_index[0].astype(jnp.int32)
    dst = edge_index[1].astype(jnp.int32)
    pad = jnp.full((EPAD - E,), NP - 1, jnp.int32)
    src2 = jnp.concatenate([src, pad]).reshape(NW, NCHUNK, K)
    dst2 = jnp.concatenate([dst, pad]).reshape(NW, NCHUNK, K)
    W3p = jnp.zeros((NFEAT, NHID), f32).at[:, :NCLASS].set(W3)
    a3sp = jnp.zeros((NHID,), f32).at[:NCLASS].set(a3_src)
    a3dp = jnp.zeros((NHID,), f32).at[:NCLASS].set(a3_dst)

    sc4 = _get_sc(NHEADS)
    sc1 = _get_sc(1)
    *ht1, sd1 = _tc1(xp, W1, a1_src, a1_dst)
    acc1 = sc4(src2, dst2, sd1, *ht1)
    *ht2, sd2 = _tc2(*acc1, W2, a2_src, a2_dst)
    acc2 = sc4(src2, dst2, sd2, *ht2)
    ht3, sd3 = _tc3(*acc2, W3p, a3sp, a3dp)
    acc3 = sc1(src2, dst2, sd3, ht3)
    out = _tc4(acc3[0])
    return out[:N, :NCLASS]


def kernel(x, edge_index, n_node_features, mini_batch,
           W1, a1_src, a1_dst, W2, a2_src, a2_dst, W3, a3_src, a3_dst):
    return _impl(x, edge_index, W1, a1_src, a1_dst,
                 W2, a2_src, a2_dst, W3, a3_src, a3_dst)


# speedup vs baseline: 27.9853x; 1.0111x over previous
import functools

import jax
import jax.numpy as jnp
from jax import lax
from jax.experimental import pallas as pl
from jax.experimental.pallas import tpu as pltpu
from jax.experimental.pallas import tpu_sc as plsc

N = 10000
E = 160000
NFEAT = 256
NHID = 64
NHEADS = 4
NCLASS = 40
NEG_SLOPE = 0.2

NP = 10240
NC = 2
NS = 16
NW = NC * NS
K = 128
NCHUNK = 40
EPW = K * NCHUNK
EPAD = NW * EPW
ACOL = 80
ROWS_PER_TILE = NP // NS
ZR = 64

NCH0 = 36
NCH1 = 44
NCHMAX = max(NCH0, NCH1)

BN = 512
GRID = NP // BN


def _elu(v):
    return jnp.where(v > 0.0, v, jnp.exp(jnp.minimum(v, 0.0)) - 1.0)


def _norm_cols(acc_refs, nheads, double_elu):
    cols = []
    for h in range(nheads):
        pa = acc_refs[h][0] + acc_refs[h][1]
        o = pa[:, :NHID] / (pa[:, NHID:NHID + 1] + 1e-16)
        o = _elu(o)
        if double_elu:
            o = _elu(o)
        cols.append(o)
    return jnp.concatenate(cols, axis=1) if len(cols) > 1 else cols[0]


def _heads_matmul(xb, w_ref, as_ref, ad_ref, nheads, ht_refs, sd_ref):
    scols = []
    dcols = []
    for h in range(nheads):
        hb = jnp.dot(xb, w_ref[h], preferred_element_type=jnp.float32)
        ht_refs[h][...] = hb
        scols.append(jnp.dot(hb, as_ref[h].reshape(NHID, 1),
                             preferred_element_type=jnp.float32))
        dcols.append(jnp.dot(hb, ad_ref[h].reshape(NHID, 1),
                             preferred_element_type=jnp.float32))
    sd_ref[...] = jnp.concatenate(scols + dcols, axis=1).T


def _make_tc1(interpret=False):
    def body(x_ref, w_ref, as_ref, ad_ref, ht0, ht1, ht2, ht3, sd_ref):
        _heads_matmul(x_ref[...], w_ref, as_ref, ad_ref, NHEADS,
                      (ht0, ht1, ht2, ht3), sd_ref)

    full = lambda shape: pl.BlockSpec(shape, lambda i: (0,) * len(shape))
    return pl.pallas_call(
        body,
        grid=(GRID,),
        in_specs=[
            pl.BlockSpec((BN, NFEAT), lambda i: (i, 0)),
            full((NHEADS, NFEAT, NHID)),
            full((NHEADS, NHID)),
            full((NHEADS, NHID)),
        ],
        out_specs=[pl.BlockSpec((BN, NHID), lambda i: (i, 0))] * NHEADS
        + [pl.BlockSpec((2 * NHEADS, BN), lambda i: (0, i))],
        out_shape=[jax.ShapeDtypeStruct((NP, NHID), jnp.float32)] * NHEADS
        + [jax.ShapeDtypeStruct((2 * NHEADS, NP), jnp.float32)],
        interpret=interpret,
    )


def _make_tc_mid(double_elu, interpret=False):
    def body(a0, a1, a2, a3, w_ref, as_ref, ad_ref, ht0, ht1, ht2, ht3, sd_ref):
        hcat = _norm_cols((a0, a1, a2, a3), NHEADS, double_elu)
        _heads_matmul(hcat, w_ref, as_ref, ad_ref, NHEADS,
                      (ht0, ht1, ht2, ht3), sd_ref)

    full = lambda shape: pl.BlockSpec(shape, lambda i: (0,) * len(shape))
    return pl.pallas_call(
        body,
        grid=(GRID,),
        in_specs=[pl.BlockSpec((NC, BN, ACOL), lambda i: (0, i, 0))] * NHEADS
        + [
            full((NHEADS, NFEAT, NHID)),
            full((NHEADS, NHID)),
            full((NHEADS, NHID)),
        ],
        out_specs=[pl.BlockSpec((BN, NHID), lambda i: (i, 0))] * NHEADS
        + [pl.BlockSpec((2 * NHEADS, BN), lambda i: (0, i))],
        out_shape=[jax.ShapeDtypeStruct((NP, NHID), jnp.float32)] * NHEADS
        + [jax.ShapeDtypeStruct((2 * NHEADS, NP), jnp.float32)],
        interpret=interpret,
    )


def _make_tc3(interpret=False):
    def body(a0, a1, a2, a3, w_ref, as_ref, ad_ref, ht_ref, sd_ref):
        hcat = _norm_cols((a0, a1, a2, a3), NHEADS, False)
        hb = jnp.dot(hcat, w_ref[...], preferred_element_type=jnp.float32)
        ht_ref[...] = hb
        s = jnp.dot(hb, as_ref[...].reshape(NHID, 1),
                    preferred_element_type=jnp.float32)
        d = jnp.dot(hb, ad_ref[...].reshape(NHID, 1),
                    preferred_element_type=jnp.float32)
        sd_ref[...] = jnp.concatenate([s, d], axis=1).T

    full = lambda shape: pl.BlockSpec(shape, lambda i: (0,) * len(shape))
    return pl.pallas_call(
        body,
        grid=(GRID,),
        in_specs=[pl.BlockSpec((NC, BN, ACOL), lambda i: (0, i, 0))] * NHEADS
        + [full((NFEAT, NHID)), full((NHID,)), full((NHID,))],
        out_specs=[
            pl.BlockSpec((BN, NHID), lambda i: (i, 0)),
            pl.BlockSpec((2, BN), lambda i: (0, i)),
        ],
        out_shape=[
            jax.ShapeDtypeStruct((NP, NHID), jnp.float32),
            jax.ShapeDtypeStruct((2, NP), jnp.float32),
        ],
        interpret=interpret,
    )


def _make_tc4(interpret=False):
    def body(a_ref, o_ref):
        pa = a_ref[0] + a_ref[1]
        o = pa[:, :NHID] / (pa[:, NHID:NHID + 1] + 1e-16)
        o_ref[...] = _elu(o)

    return pl.pallas_call(
        body,
        grid=(GRID,),
        in_specs=[pl.BlockSpec((NC, BN, ACOL), lambda i: (0, i, 0))],
        out_specs=pl.BlockSpec((BN, NHID), lambda i: (i, 0)),
        out_shape=jax.ShapeDtypeStruct((NP, NHID), jnp.float32),
        interpret=interpret,
    )


def _make_sc(nheads, interpret=False):
    mesh = plsc.VectorSubcoreMesh(core_axis_name="c", subcore_axis_name="s",
                                  num_cores=NC, num_subcores=NS)
    out_type = [jax.ShapeDtypeStruct((NC, NP, ACOL), jnp.float32)
                for _ in range(nheads)]
    scratch = [
        pltpu.VMEM((NCHMAX, K), jnp.int32),
        pltpu.VMEM((NCHMAX, K), jnp.int32),
        pltpu.VMEM((NP,), jnp.float32),
        pltpu.VMEM((NP,), jnp.float32),
        pltpu.VMEM((K,), jnp.float32),
        pltpu.VMEM((K, NHID), jnp.float32),
        pltpu.VMEM((K, NHID), jnp.float32),
        pltpu.VMEM((K, ACOL), jnp.float32),
        pltpu.VMEM((K, ACOL), jnp.float32),
        pltpu.VMEM((ZR, ACOL), jnp.float32),
        pltpu.VMEM_SHARED((NP, ACOL), jnp.float32),
        pltpu.SemaphoreType.DMA,
        pltpu.SemaphoreType.DMA,
        pltpu.SemaphoreType.DMA,
        pltpu.SemaphoreType.DMA,
        pltpu.SemaphoreType.DMA,
    ]

    @functools.partial(
        pl.kernel, out_type=out_type, mesh=mesh, scratch_types=scratch,
        compiler_params=pltpu.CompilerParams(needs_layout_passes=False,
                                             use_tc_tiling_on_sc=False),
        interpret=interpret)
    def sck(src_hbm, dst_hbm, sd_hbm, *rest):
        ht = rest[:nheads]
        outs = rest[nheads:2 * nheads]
        (src_v, dst_v, s_v, d_v, ex_v, rows_v0, rows_v1, st_v0, st_v1,
         zeros_v, acc_sh, gsem0, gsem1, zsem, ssem0, ssem1) = rest[2 * nheads:]
        cid = lax.axis_index("c")
        sid = lax.axis_index("s")
        start = jnp.where(cid == 0, sid * NCH0, NCH0 * NS + sid * NCH1)
        nch = jnp.where(cid == 0, NCH0, NCH1)
        pltpu.sync_copy(src_hbm.at[pl.ds(start, NCHMAX)], src_v)
        pltpu.sync_copy(dst_hbm.at[pl.ds(start, NCHMAX)], dst_v)
        z16 = jnp.zeros((16,), jnp.float32)
        for r in range(ZR):
            for q in range(ACOL // 16):
                zeros_v[r, pl.ds(q * 16, 16)] = z16
        iota16 = lax.iota(jnp.int32, 16)
        lanes = [jnp.full((16,), l, jnp.int32) for l in range(16)]
        row0 = sid * ROWS_PER_TILE

        def compute_ex(c):
            @plsc.parallel_loop(0, K // 16)
            def _exbody(j):
                si = src_v[c, pl.ds(j * 16, 16)]
                di = dst_v[c, pl.ds(j * 16, 16)]
                sv = plsc.load_gather(s_v, [si])
                dv = plsc.load_gather(d_v, [di])
                e = sv + dv
                e = jnp.where(e >= 0.0, e, e * NEG_SLOPE)
                ex_v[pl.ds(j * 16, 16)] = jnp.exp(e)

        def scale(rows_v, st_v):
            @plsc.parallel_loop(0, K // 16)
            def sbody(g):
                ex16 = ex_v[pl.ds(g * 16, 16)]
                base = g * 16
                for l in range(16):
                    av = ex16.at[lanes[l]].get(mode="promise_in_bounds")
                    j = base + l
                    for q in range(NHID // 16):
                        st_v[j, pl.ds(q * 16, 16)] = (
                            rows_v[j, pl.ds(q * 16, 16)] * av)
                    st_v[j, pl.ds(NHID, 16)] = jnp.where(
                        iota16 == 0, av, 0.0)

        for h in range(nheads):
            pltpu.sync_copy(sd_hbm.at[h], s_v)
            pltpu.sync_copy(sd_hbm.at[nheads + h], d_v)

            zd = [pltpu.async_copy(
                zeros_v, acc_sh.at[pl.ds(row0 + b * ZR, ZR)], zsem)
                for b in range(ROWS_PER_TILE // ZR)]
            for dsc in zd:
                dsc.wait()
            plsc.subcore_barrier()

            g0 = pltpu.async_copy(ht[h].at[src_v.at[0]], rows_v0, gsem0)

            def pair_body(i, _):
                c = 2 * i
                g1 = pltpu.async_copy(ht[h].at[src_v.at[c + 1]], rows_v1,
                                      gsem1)
                pltpu.make_async_copy(ht[h].at[src_v.at[c]], rows_v0,
                                      gsem0).wait()
                compute_ex(c)

                @pl.when(c >= 2)
                def _():
                    pltpu.make_async_copy(st_v0, acc_sh.at[dst_v.at[c]],
                                          ssem0).wait()
                scale(rows_v0, st_v0)
                pltpu.async_copy(st_v0, acc_sh.at[dst_v.at[c]], ssem0,
                                 add=True)

                @pl.when(c + 2 < nch)
                def _():
                    pltpu.async_copy(ht[h].at[src_v.at[c + 2]], rows_v0,
                                     gsem0)
                g1.wait()
                compute_ex(c + 1)

                @pl.when(c >= 2)
                def _():
                    pltpu.make_async_copy(st_v1, acc_sh.at[dst_v.at[c + 1]],
                                          ssem1).wait()
                scale(rows_v1, st_v1)
                pltpu.async_copy(st_v1, acc_sh.at[dst_v.at[c + 1]], ssem1,
                                 add=True)
                return 0
            lax.fori_loop(0, nch // 2, pair_body, 0)
            pltpu.make_async_copy(st_v0, acc_sh.at[dst_v.at[nch - 2]],
                                  ssem0).wait()
            pltpu.make_async_copy(st_v1, acc_sh.at[dst_v.at[nch - 1]],
                                  ssem1).wait()
            plsc.subcore_barrier()
            pltpu.sync_copy(
                acc_sh.at[pl.ds(row0, ROWS_PER_TILE)],
                outs[h].at[cid, pl.ds(row0, ROWS_PER_TILE)])
            plsc.subcore_barrier()

    return sck


_tc1 = _make_tc1()
_tc2 = _make_tc_mid(double_elu=True)
_tc3 = _make_tc3()
_tc4 = _make_tc4()

_SC_CACHE = {}


def _get_sc(nheads):
    if nheads not in _SC_CACHE:
        _SC_CACHE[nheads] = _make_sc(nheads)
    return _SC_CACHE[nheads]


@jax.jit
def _impl(x, edge_index, W1, a1_src, a1_dst, W2, a2_src, a2_dst,
          W3, a3_src, a3_dst):
    f32 = jnp.float32
    xp = jnp.zeros((NP, NFEAT), f32).at[:N].set(x.astype(f32))
    src = edge_index[0].astype(jnp.int32)
    dst = edge_index[1].astype(jnp.int32)
    pad = jnp.full((EPAD - E,), NP - 1, jnp.int32)
    src2 = jnp.concatenate([src, pad]).reshape(NW * NCHUNK, K)
    dst2 = jnp.concatenate([dst, pad]).reshape(NW * NCHUNK, K)
    W3p = jnp.zeros((NFEAT, NHID), f32).at[:, :NCLASS].set(W3)
    a3sp = jnp.zeros((NHID,), f32).at[:NCLASS].set(a3_src)
    a3dp = jnp.zeros((NHID,), f32).at[:NCLASS].set(a3_dst)

    sc4 = _get_sc(NHEADS)
    sc1 = _get_sc(1)
    *ht1, sd1 = _tc1(xp, W1, a1_src, a1_dst)
    acc1 = sc4(src2, dst2, sd1, *ht1)
    *ht2, sd2 = _tc2(*acc1, W2, a2_src, a2_dst)
    acc2 = sc4(src2, dst2, sd2, *ht2)
    ht3, sd3 = _tc3(*acc2, W3p, a3sp, a3dp)
    acc3 = sc1(src2, dst2, sd3, ht3)
    out = _tc4(acc3[0])
    return out[:N, :NCLASS]


def kernel(x, edge_index, n_node_features, mini_batch,
           W1, a1_src, a1_dst, W2, a2_src, a2_dst, W3, a3_src, a3_dst):
    return _impl(x, edge_index, W1, a1_src, a1_dst,
                 W2, a2_src, a2_dst, W3, a3_src, a3_dst)


# speedup vs baseline: 29.2950x; 1.0468x over previous
import functools

import jax
import jax.numpy as jnp
from jax import lax
from jax.experimental import pallas as pl
from jax.experimental.pallas import tpu as pltpu
from jax.experimental.pallas import tpu_sc as plsc

N = 10000
E = 160000
NFEAT = 256
NHID = 64
NHEADS = 4
NCLASS = 40
NEG_SLOPE = 0.2

NP = 10240
NC = 2
NS = 16
NW = NC * NS
K = 128
NCHUNK = 40
EPW = K * NCHUNK
EPAD = NW * EPW
ACOL = 80
ROWS_PER_TILE = NP // NS
ZR = 64

NCH0 = 44
NCH1 = 36
NCHMAX = max(NCH0, NCH1)

BN = 512
GRID = NP // BN


def _elu(v):
    return jnp.where(v > 0.0, v, jnp.exp(jnp.minimum(v, 0.0)) - 1.0)


def _norm_cols(acc_refs, nheads, double_elu):
    cols = []
    for h in range(nheads):
        pa = acc_refs[h][0] + acc_refs[h][1]
        o = pa[:, :NHID] / (pa[:, NHID:NHID + 1] + 1e-16)
        o = _elu(o)
        if double_elu:
            o = _elu(o)
        cols.append(o)
    return jnp.concatenate(cols, axis=1) if len(cols) > 1 else cols[0]


def _heads_matmul(xb, w_ref, as_ref, ad_ref, nheads, ht_refs, sd_ref):
    scols = []
    dcols = []
    for h in range(nheads):
        hb = jnp.dot(xb, w_ref[h], preferred_element_type=jnp.float32)
        ht_refs[h][...] = hb
        scols.append(jnp.dot(hb, as_ref[h].reshape(NHID, 1),
                             preferred_element_type=jnp.float32))
        dcols.append(jnp.dot(hb, ad_ref[h].reshape(NHID, 1),
                             preferred_element_type=jnp.float32))
    sd_ref[...] = jnp.concatenate(scols + dcols, axis=1).T


def _make_tc1(interpret=False):
    def body(x_ref, w_ref, as_ref, ad_ref, ht0, ht1, ht2, ht3, sd_ref):
        _heads_matmul(x_ref[...], w_ref, as_ref, ad_ref, NHEADS,
                      (ht0, ht1, ht2, ht3), sd_ref)

    full = lambda shape: pl.BlockSpec(shape, lambda i: (0,) * len(shape))
    return pl.pallas_call(
        body,
        grid=(GRID,),
        in_specs=[
            pl.BlockSpec((BN, NFEAT), lambda i: (i, 0)),
            full((NHEADS, NFEAT, NHID)),
            full((NHEADS, NHID)),
            full((NHEADS, NHID)),
        ],
        out_specs=[pl.BlockSpec((BN, NHID), lambda i: (i, 0))] * NHEADS
        + [pl.BlockSpec((2 * NHEADS, BN), lambda i: (0, i))],
        out_shape=[jax.ShapeDtypeStruct((NP, NHID), jnp.float32)] * NHEADS
        + [jax.ShapeDtypeStruct((2 * NHEADS, NP), jnp.float32)],
        interpret=interpret,
    )


def _make_tc_mid(double_elu, interpret=False):
    def body(a0, a1, a2, a3, w_ref, as_ref, ad_ref, ht0, ht1, ht2, ht3, sd_ref):
        hcat = _norm_cols((a0, a1, a2, a3), NHEADS, double_elu)
        _heads_matmul(hcat, w_ref, as_ref, ad_ref, NHEADS,
                      (ht0, ht1, ht2, ht3), sd_ref)

    full = lambda shape: pl.BlockSpec(shape, lambda i: (0,) * len(shape))
    return pl.pallas_call(
        body,
        grid=(GRID,),
        in_specs=[pl.BlockSpec((NC, BN, ACOL), lambda i: (0, i, 0))] * NHEADS
        + [
            full((NHEADS, NFEAT, NHID)),
            full((NHEADS, NHID)),
            full((NHEADS, NHID)),
        ],
        out_specs=[pl.BlockSpec((BN, NHID), lambda i: (i, 0))] * NHEADS
        + [pl.BlockSpec((2 * NHEADS, BN), lambda i: (0, i))],
        out_shape=[jax.ShapeDtypeStruct((NP, NHID), jnp.float32)] * NHEADS
        + [jax.ShapeDtypeStruct((2 * NHEADS, NP), jnp.float32)],
        interpret=interpret,
    )


def _make_tc3(interpret=False):
    def body(a0, a1, a2, a3, w_ref, as_ref, ad_ref, ht_ref, sd_ref):
        hcat = _norm_cols((a0, a1, a2, a3), NHEADS, False)
        hb = jnp.dot(hcat, w_ref[...], preferred_element_type=jnp.float32)
        ht_ref[...] = hb
        s = jnp.dot(hb, as_ref[...].reshape(NHID, 1),
                    preferred_element_type=jnp.float32)
        d = jnp.dot(hb, ad_ref[...].reshape(NHID, 1),
                    preferred_element_type=jnp.float32)
        sd_ref[...] = jnp.concatenate([s, d], axis=1).T

    full = lambda shape: pl.BlockSpec(shape, lambda i: (0,) * len(shape))
    return pl.pallas_call(
        body,
        grid=(GRID,),
        in_specs=[pl.BlockSpec((NC, BN, ACOL), lambda i: (0, i, 0))] * NHEADS
        + [full((NFEAT, NHID)), full((NHID,)), full((NHID,))],
        out_specs=[
            pl.BlockSpec((BN, NHID), lambda i: (i, 0)),
            pl.BlockSpec((2, BN), lambda i: (0, i)),
        ],
        out_shape=[
            jax.ShapeDtypeStruct((NP, NHID), jnp.float32),
            jax.ShapeDtypeStruct((2, NP), jnp.float32),
        ],
        interpret=interpret,
    )


def _make_tc4(interpret=False):
    def body(a_ref, o_ref):
        pa = a_ref[0] + a_ref[1]
        o = pa[:, :NHID] / (pa[:, NHID:NHID + 1] + 1e-16)
        o_ref[...] = _elu(o)

    return pl.pallas_call(
        body,
        grid=(GRID,),
        in_specs=[pl.BlockSpec((NC, BN, ACOL), lambda i: (0, i, 0))],
        out_specs=pl.BlockSpec((BN, NHID), lambda i: (i, 0)),
        out_shape=jax.ShapeDtypeStruct((NP, NHID), jnp.float32),
        interpret=interpret,
    )


def _make_sc(nheads, interpret=False):
    mesh = plsc.VectorSubcoreMesh(core_axis_name="c", subcore_axis_name="s",
                                  num_cores=NC, num_subcores=NS)
    out_type = [jax.ShapeDtypeStruct((NC, NP, ACOL), jnp.float32)
                for _ in range(nheads)]
    scratch = [
        pltpu.VMEM((NCHMAX, K), jnp.int32),
        pltpu.VMEM((NCHMAX, K), jnp.int32),
        pltpu.VMEM((NP,), jnp.float32),
        pltpu.VMEM((NP,), jnp.float32),
        pltpu.VMEM((K,), jnp.float32),
        pltpu.VMEM((K, NHID), jnp.float32),
        pltpu.VMEM((K, NHID), jnp.float32),
        pltpu.VMEM((K, ACOL), jnp.float32),
        pltpu.VMEM((K, ACOL), jnp.float32),
        pltpu.VMEM((ZR, ACOL), jnp.float32),
        pltpu.VMEM_SHARED((NP, ACOL), jnp.float32),
        pltpu.SemaphoreType.DMA,
        pltpu.SemaphoreType.DMA,
        pltpu.SemaphoreType.DMA,
        pltpu.SemaphoreType.DMA,
        pltpu.SemaphoreType.DMA,
    ]

    @functools.partial(
        pl.kernel, out_type=out_type, mesh=mesh, scratch_types=scratch,
        compiler_params=pltpu.CompilerParams(needs_layout_passes=False,
                                             use_tc_tiling_on_sc=False),
        interpret=interpret)
    def sck(src_hbm, dst_hbm, sd_hbm, *rest):
        ht = rest[:nheads]
        outs = rest[nheads:2 * nheads]
        (src_v, dst_v, s_v, d_v, ex_v, rows_v0, rows_v1, st_v0, st_v1,
         zeros_v, acc_sh, gsem0, gsem1, zsem, ssem0, ssem1) = rest[2 * nheads:]
        cid = lax.axis_index("c")
        sid = lax.axis_index("s")
        start = jnp.where(cid == 0, sid * NCH0, NCH0 * NS + sid * NCH1)
        nch = jnp.where(cid == 0, NCH0, NCH1)
        pltpu.sync_copy(src_hbm.at[pl.ds(start, NCHMAX)], src_v)
        pltpu.sync_copy(dst_hbm.at[pl.ds(start, NCHMAX)], dst_v)
        z16 = jnp.zeros((16,), jnp.float32)
        for r in range(ZR):
            for q in range(ACOL // 16):
                zeros_v[r, pl.ds(q * 16, 16)] = z16
        iota16 = lax.iota(jnp.int32, 16)
        lanes = [jnp.full((16,), l, jnp.int32) for l in range(16)]
        row0 = sid * ROWS_PER_TILE

        def compute_ex(c):
            @plsc.parallel_loop(0, K // 16)
            def _exbody(j):
                si = src_v[c, pl.ds(j * 16, 16)]
                di = dst_v[c, pl.ds(j * 16, 16)]
                sv = plsc.load_gather(s_v, [si])
                dv = plsc.load_gather(d_v, [di])
                e = sv + dv
                e = jnp.where(e >= 0.0, e, e * NEG_SLOPE)
                ex_v[pl.ds(j * 16, 16)] = jnp.exp(e)

        def scale(rows_v, st_v):
            @plsc.parallel_loop(0, K // 16)
            def sbody(g):
                ex16 = ex_v[pl.ds(g * 16, 16)]
                base = g * 16
                for l in range(16):
                    av = ex16.at[lanes[l]].get(mode="promise_in_bounds")
                    j = base + l
                    for q in range(NHID // 16):
                        st_v[j, pl.ds(q * 16, 16)] = (
                            rows_v[j, pl.ds(q * 16, 16)] * av)
                    st_v[j, pl.ds(NHID, 16)] = jnp.where(
                        iota16 == 0, av, 0.0)

        for h in range(nheads):
            pltpu.sync_copy(sd_hbm.at[h], s_v)
            pltpu.sync_copy(sd_hbm.at[nheads + h], d_v)

            zd = [pltpu.async_copy(
                zeros_v, acc_sh.at[pl.ds(row0 + b * ZR, ZR)], zsem)
                for b in range(ROWS_PER_TILE // ZR)]
            for dsc in zd:
                dsc.wait()
            plsc.subcore_barrier()

            g0 = pltpu.async_copy(ht[h].at[src_v.at[0]], rows_v0, gsem0)

            def pair_body(i, _):
                c = 2 * i
                g1 = pltpu.async_copy(ht[h].at[src_v.at[c + 1]], rows_v1,
                                      gsem1)
                pltpu.make_async_copy(ht[h].at[src_v.at[c]], rows_v0,
                                      gsem0).wait()
                compute_ex(c)

                @pl.when(c >= 2)
                def _():
                    pltpu.make_async_copy(st_v0, acc_sh.at[dst_v.at[c]],
                                          ssem0).wait()
                scale(rows_v0, st_v0)
                pltpu.async_copy(st_v0, acc_sh.at[dst_v.at[c]], ssem0,
                                 add=True)

                @pl.when(c + 2 < nch)
                def _():
                    pltpu.async_copy(ht[h].at[src_v.at[c + 2]], rows_v0,
                                     gsem0)
                g1.wait()
                compute_ex(c + 1)

                @pl.when(c >= 2)
                def _():
                    pltpu.make_async_copy(st_v1, acc_sh.at[dst_v.at[c + 1]],
                                          ssem1).wait()
                scale(rows_v1, st_v1)
                pltpu.async_copy(st_v1, acc_sh.at[dst_v.at[c + 1]], ssem1,
                                 add=True)
                return 0
            lax.fori_loop(0, nch // 2, pair_body, 0)
            pltpu.make_async_copy(st_v0, acc_sh.at[dst_v.at[nch - 2]],
                                  ssem0).wait()
            pltpu.make_async_copy(st_v1, acc_sh.at[dst_v.at[nch - 1]],
                                  ssem1).wait()
            plsc.subcore_barrier()
            pltpu.sync_copy(
                acc_sh.at[pl.ds(row0, ROWS_PER_TILE)],
                outs[h].at[cid, pl.ds(row0, ROWS_PER_TILE)])
            plsc.subcore_barrier()

    return sck


_tc1 = _make_tc1()
_tc2 = _make_tc_mid(double_elu=True)
_tc3 = _make_tc3()
_tc4 = _make_tc4()

_SC_CACHE = {}


def _get_sc(nheads):
    if nheads not in _SC_CACHE:
        _SC_CACHE[nheads] = _make_sc(nheads)
    return _SC_CACHE[nheads]


@jax.jit
def _impl(x, edge_index, W1, a1_src, a1_dst, W2, a2_src, a2_dst,
          W3, a3_src, a3_dst):
    f32 = jnp.float32
    xp = jnp.zeros((NP, NFEAT), f32).at[:N].set(x.astype(f32))
    src = edge_index[0].astype(jnp.int32)
    dst = edge_index[1].astype(jnp.int32)
    pad = jnp.full((EPAD - E,), NP - 1, jnp.int32)
    src2 = jnp.concatenate([src, pad]).reshape(NW * NCHUNK, K)
    dst2 = jnp.concatenate([dst, pad]).reshape(NW * NCHUNK, K)
    W3p = jnp.zeros((NFEAT, NHID), f32).at[:, :NCLASS].set(W3)
    a3sp = jnp.zeros((NHID,), f32).at[:NCLASS].set(a3_src)
    a3dp = jnp.zeros((NHID,), f32).at[:NCLASS].set(a3_dst)

    sc4 = _get_sc(NHEADS)
    sc1 = _get_sc(1)
    *ht1, sd1 = _tc1(xp, W1, a1_src, a1_dst)
    acc1 = sc4(src2, dst2, sd1, *ht1)
    *ht2, sd2 = _tc2(*acc1, W2, a2_src, a2_dst)
    acc2 = sc4(src2, dst2, sd2, *ht2)
    ht3, sd3 = _tc3(*acc2, W3p, a3sp, a3dp)
    acc3 = sc1(src2, dst2, sd3, ht3)
    out = _tc4(acc3[0])
    return out[:N, :NCLASS]


def kernel(x, edge_index, n_node_features, mini_batch,
           W1, a1_src, a1_dst, W2, a2_src, a2_dst, W3, a3_src, a3_dst):
    return _impl(x, edge_index, W1, a1_src, a1_dst,
                 W2, a2_src, a2_dst, W3, a3_src, a3_dst)


# speedup vs baseline: 29.7146x; 1.0143x over previous
import functools

import jax
import jax.numpy as jnp
from jax import lax
from jax.experimental import pallas as pl
from jax.experimental.pallas import tpu as pltpu
from jax.experimental.pallas import tpu_sc as plsc

N = 10000
E = 160000
NFEAT = 256
NHID = 64
NHEADS = 4
NCLASS = 40
NEG_SLOPE = 0.2

NP = 10240
NC = 2
NS = 16
NW = NC * NS
K = 128
NCHUNK = 40
EPW = K * NCHUNK
EPAD = NW * EPW
ACOL = 80
ROWS_PER_TILE = NP // NS
ZR = 64

NCH0 = 46
NCH1 = 34
NCHMAX = max(NCH0, NCH1)

BN = 512
GRID = NP // BN


def _elu(v):
    return jnp.where(v > 0.0, v, jnp.exp(jnp.minimum(v, 0.0)) - 1.0)


def _norm_cols(acc_refs, nheads, double_elu):
    cols = []
    for h in range(nheads):
        pa = acc_refs[h][0] + acc_refs[h][1]
        o = pa[:, :NHID] / (pa[:, NHID:NHID + 1] + 1e-16)
        o = _elu(o)
        if double_elu:
            o = _elu(o)
        cols.append(o)
    return jnp.concatenate(cols, axis=1) if len(cols) > 1 else cols[0]


def _heads_matmul(xb, w_ref, as_ref, ad_ref, nheads, ht_refs, sd_ref):
    scols = []
    dcols = []
    for h in range(nheads):
        hb = jnp.dot(xb, w_ref[h], preferred_element_type=jnp.float32)
        ht_refs[h][...] = hb
        scols.append(jnp.dot(hb, as_ref[h].reshape(NHID, 1),
                             preferred_element_type=jnp.float32))
        dcols.append(jnp.dot(hb, ad_ref[h].reshape(NHID, 1),
                             preferred_element_type=jnp.float32))
    sd_ref[...] = jnp.concatenate(scols + dcols, axis=1).T


def _make_tc1(interpret=False):
    def body(x_ref, w_ref, as_ref, ad_ref, ht0, ht1, ht2, ht3, sd_ref):
        _heads_matmul(x_ref[...], w_ref, as_ref, ad_ref, NHEADS,
                      (ht0, ht1, ht2, ht3), sd_ref)

    full = lambda shape: pl.BlockSpec(shape, lambda i: (0,) * len(shape))
    return pl.pallas_call(
        body,
        grid=(GRID,),
        in_specs=[
            pl.BlockSpec((BN, NFEAT), lambda i: (i, 0)),
            full((NHEADS, NFEAT, NHID)),
            full((NHEADS, NHID)),
            full((NHEADS, NHID)),
        ],
        out_specs=[pl.BlockSpec((BN, NHID), lambda i: (i, 0))] * NHEADS
        + [pl.BlockSpec((2 * NHEADS, BN), lambda i: (0, i))],
        out_shape=[jax.ShapeDtypeStruct((NP, NHID), jnp.float32)] * NHEADS
        + [jax.ShapeDtypeStruct((2 * NHEADS, NP), jnp.float32)],
        interpret=interpret,
    )


def _make_tc_mid(double_elu, interpret=False):
    def body(a0, a1, a2, a3, w_ref, as_ref, ad_ref, ht0, ht1, ht2, ht3, sd_ref):
        hcat = _norm_cols((a0, a1, a2, a3), NHEADS, double_elu)
        _heads_matmul(hcat, w_ref, as_ref, ad_ref, NHEADS,
                      (ht0, ht1, ht2, ht3), sd_ref)

    full = lambda shape: pl.BlockSpec(shape, lambda i: (0,) * len(shape))
    return pl.pallas_call(
        body,
        grid=(GRID,),
        in_specs=[pl.BlockSpec((NC, BN, ACOL), lambda i: (0, i, 0))] * NHEADS
        + [
            full((NHEADS, NFEAT, NHID)),
            full((NHEADS, NHID)),
            full((NHEADS, NHID)),
        ],
        out_specs=[pl.BlockSpec((BN, NHID), lambda i: (i, 0))] * NHEADS
        + [pl.BlockSpec((2 * NHEADS, BN), lambda i: (0, i))],
        out_shape=[jax.ShapeDtypeStruct((NP, NHID), jnp.float32)] * NHEADS
        + [jax.ShapeDtypeStruct((2 * NHEADS, NP), jnp.float32)],
        interpret=interpret,
    )


def _make_tc3(interpret=False):
    def body(a0, a1, a2, a3, w_ref, as_ref, ad_ref, ht_ref, sd_ref):
        hcat = _norm_cols((a0, a1, a2, a3), NHEADS, False)
        hb = jnp.dot(hcat, w_ref[...], preferred_element_type=jnp.float32)
        ht_ref[...] = hb
        s = jnp.dot(hb, as_ref[...].reshape(NHID, 1),
                    preferred_element_type=jnp.float32)
        d = jnp.dot(hb, ad_ref[...].reshape(NHID, 1),
                    preferred_element_type=jnp.float32)
        sd_ref[...] = jnp.concatenate([s, d], axis=1).T

    full = lambda shape: pl.BlockSpec(shape, lambda i: (0,) * len(shape))
    return pl.pallas_call(
        body,
        grid=(GRID,),
        in_specs=[pl.BlockSpec((NC, BN, ACOL), lambda i: (0, i, 0))] * NHEADS
        + [full((NFEAT, NHID)), full((NHID,)), full((NHID,))],
        out_specs=[
            pl.BlockSpec((BN, NHID), lambda i: (i, 0)),
            pl.BlockSpec((2, BN), lambda i: (0, i)),
        ],
        out_shape=[
            jax.ShapeDtypeStruct((NP, NHID), jnp.float32),
            jax.ShapeDtypeStruct((2, NP), jnp.float32),
        ],
        interpret=interpret,
    )


def _make_tc4(interpret=False):
    def body(a_ref, o_ref):
        pa = a_ref[0] + a_ref[1]
        o = pa[:, :NHID] / (pa[:, NHID:NHID + 1] + 1e-16)
        o_ref[...] = _elu(o)

    return pl.pallas_call(
        body,
        grid=(GRID,),
        in_specs=[pl.BlockSpec((NC, BN, ACOL), lambda i: (0, i, 0))],
        out_specs=pl.BlockSpec((BN, NHID), lambda i: (i, 0)),
        out_shape=jax.ShapeDtypeStruct((NP, NHID), jnp.float32),
        interpret=interpret,
    )


def _make_sc(nheads, interpret=False):
    mesh = plsc.VectorSubcoreMesh(core_axis_name="c", subcore_axis_name="s",
                                  num_cores=NC, num_subcores=NS)
    out_type = [jax.ShapeDtypeStruct((NC, NP, ACOL), jnp.float32)
                for _ in range(nheads)]
    scratch = [
        pltpu.VMEM((NCHMAX, K), jnp.int32),
        pltpu.VMEM((NCHMAX, K), jnp.int32),
        pltpu.VMEM((NP,), jnp.float32),
        pltpu.VMEM((NP,), jnp.float32),
        pltpu.VMEM((K,), jnp.float32),
        pltpu.VMEM((K, NHID), jnp.float32),
        pltpu.VMEM((K, NHID), jnp.float32),
        pltpu.VMEM((K, ACOL), jnp.float32),
        pltpu.VMEM((K, ACOL), jnp.float32),
        pltpu.VMEM((ZR, ACOL), jnp.float32),
        pltpu.VMEM_SHARED((NP, ACOL), jnp.float32),
        pltpu.SemaphoreType.DMA,
        pltpu.SemaphoreType.DMA,
        pltpu.SemaphoreType.DMA,
        pltpu.SemaphoreType.DMA,
        pltpu.SemaphoreType.DMA,
    ]

    @functools.partial(
        pl.kernel, out_type=out_type, mesh=mesh, scratch_types=scratch,
        compiler_params=pltpu.CompilerParams(needs_layout_passes=False,
                                             use_tc_tiling_on_sc=False),
        interpret=interpret)
    def sck(src_hbm, dst_hbm, sd_hbm, *rest):
        ht = rest[:nheads]
        outs = rest[nheads:2 * nheads]
        (src_v, dst_v, s_v, d_v, ex_v, rows_v0, rows_v1, st_v0, st_v1,
         zeros_v, acc_sh, gsem0, gsem1, zsem, ssem0, ssem1) = rest[2 * nheads:]
        cid = lax.axis_index("c")
        sid = lax.axis_index("s")
        start = jnp.where(cid == 0, sid * NCH0, NCH0 * NS + sid * NCH1)
        nch = jnp.where(cid == 0, NCH0, NCH1)
        pltpu.sync_copy(src_hbm.at[pl.ds(start, NCHMAX)], src_v)
        pltpu.sync_copy(dst_hbm.at[pl.ds(start, NCHMAX)], dst_v)
        z16 = jnp.zeros((16,), jnp.float32)
        for r in range(ZR):
            for q in range(ACOL // 16):
                zeros_v[r, pl.ds(q * 16, 16)] = z16
        iota16 = lax.iota(jnp.int32, 16)
        lanes = [jnp.full((16,), l, jnp.int32) for l in range(16)]
        row0 = sid * ROWS_PER_TILE

        def compute_ex(c):
            @plsc.parallel_loop(0, K // 16)
            def _exbody(j):
                si = src_v[c, pl.ds(j * 16, 16)]
                di = dst_v[c, pl.ds(j * 16, 16)]
                sv = plsc.load_gather(s_v, [si])
                dv = plsc.load_gather(d_v, [di])
                e = sv + dv
                e = jnp.where(e >= 0.0, e, e * NEG_SLOPE)
                ex_v[pl.ds(j * 16, 16)] = jnp.exp(e)

        def scale(rows_v, st_v):
            @plsc.parallel_loop(0, K // 16)
            def sbody(g):
                ex16 = ex_v[pl.ds(g * 16, 16)]
                base = g * 16
                for l in range(16):
                    av = ex16.at[lanes[l]].get(mode="promise_in_bounds")
                    j = base + l
                    for q in range(NHID // 16):
                        st_v[j, pl.ds(q * 16, 16)] = (
                            rows_v[j, pl.ds(q * 16, 16)] * av)
                    st_v[j, pl.ds(NHID, 16)] = jnp.where(
                        iota16 == 0, av, 0.0)

        for h in range(nheads):
            pltpu.sync_copy(sd_hbm.at[h], s_v)
            pltpu.sync_copy(sd_hbm.at[nheads + h], d_v)

            zd = [pltpu.async_copy(
                zeros_v, acc_sh.at[pl.ds(row0 + b * ZR, ZR)], zsem)
                for b in range(ROWS_PER_TILE // ZR)]
            for dsc in zd:
                dsc.wait()
            plsc.subcore_barrier()

            g0 = pltpu.async_copy(ht[h].at[src_v.at[0]], rows_v0, gsem0)

            def pair_body(i, _):
                c = 2 * i
                g1 = pltpu.async_copy(ht[h].at[src_v.at[c + 1]], rows_v1,
                                      gsem1)
                pltpu.make_async_copy(ht[h].at[src_v.at[c]], rows_v0,
                                      gsem0).wait()
                compute_ex(c)

                @pl.when(c >= 2)
                def _():
                    pltpu.make_async_copy(st_v0, acc_sh.at[dst_v.at[c]],
                                          ssem0).wait()
                scale(rows_v0, st_v0)
                pltpu.async_copy(st_v0, acc_sh.at[dst_v.at[c]], ssem0,
                                 add=True)

                @pl.when(c + 2 < nch)
                def _():
                    pltpu.async_copy(ht[h].at[src_v.at[c + 2]], rows_v0,
                                     gsem0)
                g1.wait()
                compute_ex(c + 1)

                @pl.when(c >= 2)
                def _():
                    pltpu.make_async_copy(st_v1, acc_sh.at[dst_v.at[c + 1]],
                                          ssem1).wait()
                scale(rows_v1, st_v1)
                pltpu.async_copy(st_v1, acc_sh.at[dst_v.at[c + 1]], ssem1,
                                 add=True)
                return 0
            lax.fori_loop(0, nch // 2, pair_body, 0)
            pltpu.make_async_copy(st_v0, acc_sh.at[dst_v.at[nch - 2]],
                                  ssem0).wait()
            pltpu.make_async_copy(st_v1, acc_sh.at[dst_v.at[nch - 1]],
                                  ssem1).wait()
            plsc.subcore_barrier()
            pltpu.sync_copy(
                acc_sh.at[pl.ds(row0, ROWS_PER_TILE)],
                outs[h].at[cid, pl.ds(row0, ROWS_PER_TILE)])
            plsc.subcore_barrier()

    return sck


_tc1 = _make_tc1()
_tc2 = _make_tc_mid(double_elu=True)
_tc3 = _make_tc3()
_tc4 = _make_tc4()

_SC_CACHE = {}


def _get_sc(nheads):
    if nheads not in _SC_CACHE:
        _SC_CACHE[nheads] = _make_sc(nheads)
    return _SC_CACHE[nheads]


@jax.jit
def _impl(x, edge_index, W1, a1_src, a1_dst, W2, a2_src, a2_dst,
          W3, a3_src, a3_dst):
    f32 = jnp.float32
    xp = jnp.zeros((NP, NFEAT), f32).at[:N].set(x.astype(f32))
    src = edge_index[0].astype(jnp.int32)
    dst = edge_index[1].astype(jnp.int32)
    pad = jnp.full((EPAD - E,), NP - 1, jnp.int32)
    src2 = jnp.concatenate([src, pad]).reshape(NW * NCHUNK, K)
    dst2 = jnp.concatenate([dst, pad]).reshape(NW * NCHUNK, K)
    W3p = jnp.zeros((NFEAT, NHID), f32).at[:, :NCLASS].set(W3)
    a3sp = jnp.zeros((NHID,), f32).at[:NCLASS].set(a3_src)
    a3dp = jnp.zeros((NHID,), f32).at[:NCLASS].set(a3_dst)

    sc4 = _get_sc(NHEADS)
    sc1 = _get_sc(1)
    *ht1, sd1 = _tc1(xp, W1, a1_src, a1_dst)
    acc1 = sc4(src2, dst2, sd1, *ht1)
    *ht2, sd2 = _tc2(*acc1, W2, a2_src, a2_dst)
    acc2 = sc4(src2, dst2, sd2, *ht2)
    ht3, sd3 = _tc3(*acc2, W3p, a3sp, a3dp)
    acc3 = sc1(src2, dst2, sd3, ht3)
    out = _tc4(acc3[0])
    return out[:N, :NCLASS]


def kernel(x, edge_index, n_node_features, mini_batch,
           W1, a1_src, a1_dst, W2, a2_src, a2_dst, W3, a3_src, a3_dst):
    return _impl(x, edge_index, W1, a1_src, a1_dst,
                 W2, a2_src, a2_dst, W3, a3_src, a3_dst)


# speedup vs baseline: 30.0479x; 1.0112x over previous
import functools

import jax
import jax.numpy as jnp
from jax import lax
from jax.experimental import pallas as pl
from jax.experimental.pallas import tpu as pltpu
from jax.experimental.pallas import tpu_sc as plsc

N = 10000
E = 160000
NFEAT = 256
NHID = 64
NHEADS = 4
NCLASS = 40
NEG_SLOPE = 0.2

NP = 10240
NC = 2
NS = 16
NW = NC * NS
K = 128
NCHUNK = 40
EPW = K * NCHUNK
EPAD = NW * EPW
ACOL = 80
ROWS_PER_TILE = NP // NS
ZR = 64

NCH0 = 48
NCH1 = 32
NCHMAX = max(NCH0, NCH1)

BN = 512
GRID = NP // BN


def _elu(v):
    return jnp.where(v > 0.0, v, jnp.exp(jnp.minimum(v, 0.0)) - 1.0)


def _norm_cols(acc_refs, nheads, double_elu):
    cols = []
    for h in range(nheads):
        pa = acc_refs[h][0] + acc_refs[h][1]
        o = pa[:, :NHID] / (pa[:, NHID:NHID + 1] + 1e-16)
        o = _elu(o)
        if double_elu:
            o = _elu(o)
        cols.append(o)
    return jnp.concatenate(cols, axis=1) if len(cols) > 1 else cols[0]


def _heads_matmul(xb, w_ref, as_ref, ad_ref, nheads, ht_refs, sd_ref):
    scols = []
    dcols = []
    for h in range(nheads):
        hb = jnp.dot(xb, w_ref[h], preferred_element_type=jnp.float32)
        ht_refs[h][...] = hb
        scols.append(jnp.dot(hb, as_ref[h].reshape(NHID, 1),
                             preferred_element_type=jnp.float32))
        dcols.append(jnp.dot(hb, ad_ref[h].reshape(NHID, 1),
                             preferred_element_type=jnp.float32))
    sd_ref[...] = jnp.concatenate(scols + dcols, axis=1).T


def _make_tc1(interpret=False):
    def body(x_ref, w_ref, as_ref, ad_ref, ht0, ht1, ht2, ht3, sd_ref):
        _heads_matmul(x_ref[...], w_ref, as_ref, ad_ref, NHEADS,
                      (ht0, ht1, ht2, ht3), sd_ref)

    full = lambda shape: pl.BlockSpec(shape, lambda i: (0,) * len(shape))
    return pl.pallas_call(
        body,
        grid=(GRID,),
        in_specs=[
            pl.BlockSpec((BN, NFEAT), lambda i: (i, 0)),
            full((NHEADS, NFEAT, NHID)),
            full((NHEADS, NHID)),
            full((NHEADS, NHID)),
        ],
        out_specs=[pl.BlockSpec((BN, NHID), lambda i: (i, 0))] * NHEADS
        + [pl.BlockSpec((2 * NHEADS, BN), lambda i: (0, i))],
        out_shape=[jax.ShapeDtypeStruct((NP, NHID), jnp.float32)] * NHEADS
        + [jax.ShapeDtypeStruct((2 * NHEADS, NP), jnp.float32)],
        interpret=interpret,
    )


def _make_tc_mid(double_elu, interpret=False):
    def body(a0, a1, a2, a3, w_ref, as_ref, ad_ref, ht0, ht1, ht2, ht3, sd_ref):
        hcat = _norm_cols((a0, a1, a2, a3), NHEADS, double_elu)
        _heads_matmul(hcat, w_ref, as_ref, ad_ref, NHEADS,
                      (ht0, ht1, ht2, ht3), sd_ref)

    full = lambda shape: pl.BlockSpec(shape, lambda i: (0,) * len(shape))
    return pl.pallas_call(
        body,
        grid=(GRID,),
        in_specs=[pl.BlockSpec((NC, BN, ACOL), lambda i: (0, i, 0))] * NHEADS
        + [
            full((NHEADS, NFEAT, NHID)),
            full((NHEADS, NHID)),
            full((NHEADS, NHID)),
        ],
        out_specs=[pl.BlockSpec((BN, NHID), lambda i: (i, 0))] * NHEADS
        + [pl.BlockSpec((2 * NHEADS, BN), lambda i: (0, i))],
        out_shape=[jax.ShapeDtypeStruct((NP, NHID), jnp.float32)] * NHEADS
        + [jax.ShapeDtypeStruct((2 * NHEADS, NP), jnp.float32)],
        interpret=interpret,
    )


def _make_tc3(interpret=False):
    def body(a0, a1, a2, a3, w_ref, as_ref, ad_ref, ht_ref, sd_ref):
        hcat = _norm_cols((a0, a1, a2, a3), NHEADS, False)
        hb = jnp.dot(hcat, w_ref[...], preferred_element_type=jnp.float32)
        ht_ref[...] = hb
        s = jnp.dot(hb, as_ref[...].reshape(NHID, 1),
                    preferred_element_type=jnp.float32)
        d = jnp.dot(hb, ad_ref[...].reshape(NHID, 1),
                    preferred_element_type=jnp.float32)
        sd_ref[...] = jnp.concatenate([s, d], axis=1).T

    full = lambda shape: pl.BlockSpec(shape, lambda i: (0,) * len(shape))
    return pl.pallas_call(
        body,
        grid=(GRID,),
        in_specs=[pl.BlockSpec((NC, BN, ACOL), lambda i: (0, i, 0))] * NHEADS
        + [full((NFEAT, NHID)), full((NHID,)), full((NHID,))],
        out_specs=[
            pl.BlockSpec((BN, NHID), lambda i: (i, 0)),
            pl.BlockSpec((2, BN), lambda i: (0, i)),
        ],
        out_shape=[
            jax.ShapeDtypeStruct((NP, NHID), jnp.float32),
            jax.ShapeDtypeStruct((2, NP), jnp.float32),
        ],
        interpret=interpret,
    )


def _make_tc4(interpret=False):
    def body(a_ref, o_ref):
        pa = a_ref[0] + a_ref[1]
        o = pa[:, :NHID] / (pa[:, NHID:NHID + 1] + 1e-16)
        o_ref[...] = _elu(o)

    return pl.pallas_call(
        body,
        grid=(GRID,),
        in_specs=[pl.BlockSpec((NC, BN, ACOL), lambda i: (0, i, 0))],
        out_specs=pl.BlockSpec((BN, NHID), lambda i: (i, 0)),
        out_shape=jax.ShapeDtypeStruct((NP, NHID), jnp.float32),
        interpret=interpret,
    )


def _make_sc(nheads, interpret=False):
    mesh = plsc.VectorSubcoreMesh(core_axis_name="c", subcore_axis_name="s",
                                  num_cores=NC, num_subcores=NS)
    out_type = [jax.ShapeDtypeStruct((NC, NP, ACOL), jnp.float32)
                for _ in range(nheads)]
    scratch = [
        pltpu.VMEM((NCHMAX, K), jnp.int32),
        pltpu.VMEM((NCHMAX, K), jnp.int32),
        pltpu.VMEM((NP,), jnp.float32),
        pltpu.VMEM((NP,), jnp.float32),
        pltpu.VMEM((K,), jnp.float32),
        pltpu.VMEM((K, NHID), jnp.float32),
        pltpu.VMEM((K, NHID), jnp.float32),
        pltpu.VMEM((K, ACOL), jnp.float32),
        pltpu.VMEM((K, ACOL), jnp.float32),
        pltpu.VMEM((ZR, ACOL), jnp.float32),
        pltpu.VMEM_SHARED((NP, ACOL), jnp.float32),
        pltpu.SemaphoreType.DMA,
        pltpu.SemaphoreType.DMA,
        pltpu.SemaphoreType.DMA,
        pltpu.SemaphoreType.DMA,
        pltpu.SemaphoreType.DMA,
    ]

    @functools.partial(
        pl.kernel, out_type=out_type, mesh=mesh, scratch_types=scratch,
        compiler_params=pltpu.CompilerParams(needs_layout_passes=False,
                                             use_tc_tiling_on_sc=False),
        interpret=interpret)
    def sck(src_hbm, dst_hbm, sd_hbm, *rest):
        ht = rest[:nheads]
        outs = rest[nheads:2 * nheads]
        (src_v, dst_v, s_v, d_v, ex_v, rows_v0, rows_v1, st_v0, st_v1,
         zeros_v, acc_sh, gsem0, gsem1, zsem, ssem0, ssem1) = rest[2 * nheads:]
        cid = lax.axis_index("c")
        sid = lax.axis_index("s")
        start = jnp.where(cid == 0, sid * NCH0, NCH0 * NS + sid * NCH1)
        nch = jnp.where(cid == 0, NCH0, NCH1)
        pltpu.sync_copy(src_hbm.at[pl.ds(start, NCHMAX)], src_v)
        pltpu.sync_copy(dst_hbm.at[pl.ds(start, NCHMAX)], dst_v)
        z16 = jnp.zeros((16,), jnp.float32)
        for r in range(ZR):
            for q in range(ACOL // 16):
                zeros_v[r, pl.ds(q * 16, 16)] = z16
        iota16 = lax.iota(jnp.int32, 16)
        lanes = [jnp.full((16,), l, jnp.int32) for l in range(16)]
        row0 = sid * ROWS_PER_TILE

        def compute_ex(c):
            @plsc.parallel_loop(0, K // 16)
            def _exbody(j):
                si = src_v[c, pl.ds(j * 16, 16)]
                di = dst_v[c, pl.ds(j * 16, 16)]
                sv = plsc.load_gather(s_v, [si])
                dv = plsc.load_gather(d_v, [di])
                e = sv + dv
                e = jnp.where(e >= 0.0, e, e * NEG_SLOPE)
                ex_v[pl.ds(j * 16, 16)] = jnp.exp(e)

        def scale(rows_v, st_v):
            @plsc.parallel_loop(0, K // 16)
            def sbody(g):
                ex16 = ex_v[pl.ds(g * 16, 16)]
                base = g * 16
                for l in range(16):
                    av = ex16.at[lanes[l]].get(mode="promise_in_bounds")
                    j = base + l
                    for q in range(NHID // 16):
                        st_v[j, pl.ds(q * 16, 16)] = (
                            rows_v[j, pl.ds(q * 16, 16)] * av)
                    st_v[j, pl.ds(NHID, 16)] = jnp.where(
                        iota16 == 0, av, 0.0)

        for h in range(nheads):
            pltpu.sync_copy(sd_hbm.at[h], s_v)
            pltpu.sync_copy(sd_hbm.at[nheads + h], d_v)

            zd = [pltpu.async_copy(
                zeros_v, acc_sh.at[pl.ds(row0 + b * ZR, ZR)], zsem)
                for b in range(ROWS_PER_TILE // ZR)]
            for dsc in zd:
                dsc.wait()
            plsc.subcore_barrier()

            g0 = pltpu.async_copy(ht[h].at[src_v.at[0]], rows_v0, gsem0)

            def pair_body(i, _):
                c = 2 * i
                g1 = pltpu.async_copy(ht[h].at[src_v.at[c + 1]], rows_v1,
                                      gsem1)
                pltpu.make_async_copy(ht[h].at[src_v.at[c]], rows_v0,
                                      gsem0).wait()
                compute_ex(c)

                @pl.when(c >= 2)
                def _():
                    pltpu.make_async_copy(st_v0, acc_sh.at[dst_v.at[c]],
                                          ssem0).wait()
                scale(rows_v0, st_v0)
                pltpu.async_copy(st_v0, acc_sh.at[dst_v.at[c]], ssem0,
                                 add=True)

                @pl.when(c + 2 < nch)
                def _():
                    pltpu.async_copy(ht[h].at[src_v.at[c + 2]], rows_v0,
                                     gsem0)
                g1.wait()
                compute_ex(c + 1)

                @pl.when(c >= 2)
                def _():
                    pltpu.make_async_copy(st_v1, acc_sh.at[dst_v.at[c + 1]],
                                          ssem1).wait()
                scale(rows_v1, st_v1)
                pltpu.async_copy(st_v1, acc_sh.at[dst_v.at[c + 1]], ssem1,
                                 add=True)
                return 0
            lax.fori_loop(0, nch // 2, pair_body, 0)
            pltpu.make_async_copy(st_v0, acc_sh.at[dst_v.at[nch - 2]],
                                  ssem0).wait()
            pltpu.make_async_copy(st_v1, acc_sh.at[dst_v.at[nch - 1]],
                                  ssem1).wait()
            plsc.subcore_barrier()
            pltpu.sync_copy(
                acc_sh.at[pl.ds(row0, ROWS_PER_TILE)],
                outs[h].at[cid, pl.ds(row0, ROWS_PER_TILE)])
            plsc.subcore_barrier()

    return sck


_tc1 = _make_tc1()
_tc2 = _make_tc_mid(double_elu=True)
_tc3 = _make_tc3()
_tc4 = _make_tc4()

_SC_CACHE = {}


def _get_sc(nheads):
    if nheads not in _SC_CACHE:
        _SC_CACHE[nheads] = _make_sc(nheads)
    return _SC_CACHE[nheads]


@jax.jit
def _impl(x, edge_index, W1, a1_src, a1_dst, W2, a2_src, a2_dst,
          W3, a3_src, a3_dst):
    f32 = jnp.float32
    xp = jnp.zeros((NP, NFEAT), f32).at[:N].set(x.astype(f32))
    src = edge_index[0].astype(jnp.int32)
    dst = edge_index[1].astype(jnp.int32)
    pad = jnp.full((EPAD - E,), NP - 1, jnp.int32)
    src2 = jnp.concatenate([src, pad]).reshape(NW * NCHUNK, K)
    dst2 = jnp.concatenate([dst, pad]).reshape(NW * NCHUNK, K)
    W3p = jnp.zeros((NFEAT, NHID), f32).at[:, :NCLASS].set(W3)
    a3sp = jnp.zeros((NHID,), f32).at[:NCLASS].set(a3_src)
    a3dp = jnp.zeros((NHID,), f32).at[:NCLASS].set(a3_dst)

    sc4 = _get_sc(NHEADS)
    sc1 = _get_sc(1)
    *ht1, sd1 = _tc1(xp, W1, a1_src, a1_dst)
    acc1 = sc4(src2, dst2, sd1, *ht1)
    *ht2, sd2 = _tc2(*acc1, W2, a2_src, a2_dst)
    acc2 = sc4(src2, dst2, sd2, *ht2)
    ht3, sd3 = _tc3(*acc2, W3p, a3sp, a3dp)
    acc3 = sc1(src2, dst2, sd3, ht3)
    out = _tc4(acc3[0])
    return out[:N, :NCLASS]


def kernel(x, edge_index, n_node_features, mini_batch,
           W1, a1_src, a1_dst, W2, a2_src, a2_dst, W3, a3_src, a3_dst):
    return _impl(x, edge_index, W1, a1_src, a1_dst,
                 W2, a2_src, a2_dst, W3, a3_src, a3_dst)


# speedup vs baseline: 30.1149x; 1.0022x over previous
import functools

import jax
import jax.numpy as jnp
from jax import lax
from jax.experimental import pallas as pl
from jax.experimental.pallas import tpu as pltpu
from jax.experimental.pallas import tpu_sc as plsc

N = 10000
E = 160000
NFEAT = 256
NHID = 64
NHEADS = 4
NCLASS = 40
NEG_SLOPE = 0.2

NP = 10240
NC = 2
NS = 16
NW = NC * NS
K = 128
NCHUNK = 40
EPW = K * NCHUNK
EPAD = NW * EPW
ACOL = 80
ROWS_PER_TILE = NP // NS
ZR = 64

NCH0 = 50
NCH1 = 30
NCHMAX = max(NCH0, NCH1)

BN = 512
GRID = NP // BN


def _elu(v):
    return jnp.where(v > 0.0, v, jnp.exp(jnp.minimum(v, 0.0)) - 1.0)


def _norm_cols(acc_refs, nheads, double_elu):
    cols = []
    for h in range(nheads):
        pa = acc_refs[h][0] + acc_refs[h][1]
        o = pa[:, :NHID] / (pa[:, NHID:NHID + 1] + 1e-16)
        o = _elu(o)
        if double_elu:
            o = _elu(o)
        cols.append(o)
    return jnp.concatenate(cols, axis=1) if len(cols) > 1 else cols[0]


def _heads_matmul(xb, w_ref, as_ref, ad_ref, nheads, ht_refs, sd_ref):
    scols = []
    dcols = []
    for h in range(nheads):
        hb = jnp.dot(xb, w_ref[h], preferred_element_type=jnp.float32)
        ht_refs[h][...] = hb
        scols.append(jnp.dot(hb, as_ref[h].reshape(NHID, 1),
                             preferred_element_type=jnp.float32))
        dcols.append(jnp.dot(hb, ad_ref[h].reshape(NHID, 1),
                             preferred_element_type=jnp.float32))
    sd_ref[...] = jnp.concatenate(scols + dcols, axis=1).T


def _make_tc1(interpret=False):
    def body(x_ref, w_ref, as_ref, ad_ref, ht0, ht1, ht2, ht3, sd_ref):
        _heads_matmul(x_ref[...], w_ref, as_ref, ad_ref, NHEADS,
                      (ht0, ht1, ht2, ht3), sd_ref)

    full = lambda shape: pl.BlockSpec(shape, lambda i: (0,) * len(shape))
    return pl.pallas_call(
        body,
        grid=(GRID,),
        in_specs=[
            pl.BlockSpec((BN, NFEAT), lambda i: (i, 0)),
            full((NHEADS, NFEAT, NHID)),
            full((NHEADS, NHID)),
            full((NHEADS, NHID)),
        ],
        out_specs=[pl.BlockSpec((BN, NHID), lambda i: (i, 0))] * NHEADS
        + [pl.BlockSpec((2 * NHEADS, BN), lambda i: (0, i))],
        out_shape=[jax.ShapeDtypeStruct((NP, NHID), jnp.float32)] * NHEADS
        + [jax.ShapeDtypeStruct((2 * NHEADS, NP), jnp.float32)],
        interpret=interpret,
    )


def _make_tc_mid(double_elu, interpret=False):
    def body(a0, a1, a2, a3, w_ref, as_ref, ad_ref, ht0, ht1, ht2, ht3, sd_ref):
        hcat = _norm_cols((a0, a1, a2, a3), NHEADS, double_elu)
        _heads_matmul(hcat, w_ref, as_ref, ad_ref, NHEADS,
                      (ht0, ht1, ht2, ht3), sd_ref)

    full = lambda shape: pl.BlockSpec(shape, lambda i: (0,) * len(shape))
    return pl.pallas_call(
        body,
        grid=(GRID,),
        in_specs=[pl.BlockSpec((NC, BN, ACOL), lambda i: (0, i, 0))] * NHEADS
        + [
            full((NHEADS, NFEAT, NHID)),
            full((NHEADS, NHID)),
            full((NHEADS, NHID)),
        ],
        out_specs=[pl.BlockSpec((BN, NHID), lambda i: (i, 0))] * NHEADS
        + [pl.BlockSpec((2 * NHEADS, BN), lambda i: (0, i))],
        out_shape=[jax.ShapeDtypeStruct((NP, NHID), jnp.float32)] * NHEADS
        + [jax.ShapeDtypeStruct((2 * NHEADS, NP), jnp.float32)],
        interpret=interpret,
    )


def _make_tc3(interpret=False):
    def body(a0, a1, a2, a3, w_ref, as_ref, ad_ref, ht_ref, sd_ref):
        hcat = _norm_cols((a0, a1, a2, a3), NHEADS, False)
        hb = jnp.dot(hcat, w_ref[...], preferred_element_type=jnp.float32)
        ht_ref[...] = hb
        s = jnp.dot(hb, as_ref[...].reshape(NHID, 1),
                    preferred_element_type=jnp.float32)
        d = jnp.dot(hb, ad_ref[...].reshape(NHID, 1),
                    preferred_element_type=jnp.float32)
        sd_ref[...] = jnp.concatenate([s, d], axis=1).T

    full = lambda shape: pl.BlockSpec(shape, lambda i: (0,) * len(shape))
    return pl.pallas_call(
        body,
        grid=(GRID,),
        in_specs=[pl.BlockSpec((NC, BN, ACOL), lambda i: (0, i, 0))] * NHEADS
        + [full((NFEAT, NHID)), full((NHID,)), full((NHID,))],
        out_specs=[
            pl.BlockSpec((BN, NHID), lambda i: (i, 0)),
            pl.BlockSpec((2, BN), lambda i: (0, i)),
        ],
        out_shape=[
            jax.ShapeDtypeStruct((NP, NHID), jnp.float32),
            jax.ShapeDtypeStruct((2, NP), jnp.float32),
        ],
        interpret=interpret,
    )


def _make_tc4(interpret=False):
    def body(a_ref, o_ref):
        pa = a_ref[0] + a_ref[1]
        o = pa[:, :NHID] / (pa[:, NHID:NHID + 1] + 1e-16)
        o_ref[...] = _elu(o)

    return pl.pallas_call(
        body,
        grid=(GRID,),
        in_specs=[pl.BlockSpec((NC, BN, ACOL), lambda i: (0, i, 0))],
        out_specs=pl.BlockSpec((BN, NHID), lambda i: (i, 0)),
        out_shape=jax.ShapeDtypeStruct((NP, NHID), jnp.float32),
        interpret=interpret,
    )


def _make_sc(nheads, interpret=False):
    mesh = plsc.VectorSubcoreMesh(core_axis_name="c", subcore_axis_name="s",
                                  num_cores=NC, num_subcores=NS)
    out_type = [jax.ShapeDtypeStruct((NC, NP, ACOL), jnp.float32)
                for _ in range(nheads)]
    scratch = [
        pltpu.VMEM((NCHMAX, K), jnp.int32),
        pltpu.VMEM((NCHMAX, K), jnp.int32),
        pltpu.VMEM((NP,), jnp.float32),
        pltpu.VMEM((NP,), jnp.float32),
        pltpu.VMEM((K,), jnp.float32),
        pltpu.VMEM((K, NHID), jnp.float32),
        pltpu.VMEM((K, NHID), jnp.float32),
        pltpu.VMEM((K, ACOL), jnp.float32),
        pltpu.VMEM((K, ACOL), jnp.float32),
        pltpu.VMEM((ZR, ACOL), jnp.float32),
        pltpu.VMEM_SHARED((NP, ACOL), jnp.float32),
        pltpu.SemaphoreType.DMA,
        pltpu.SemaphoreType.DMA,
        pltpu.SemaphoreType.DMA,
        pltpu.SemaphoreType.DMA,
        pltpu.SemaphoreType.DMA,
    ]

    @functools.partial(
        pl.kernel, out_type=out_type, mesh=mesh, scratch_types=scratch,
        compiler_params=pltpu.CompilerParams(needs_layout_passes=False,
                                             use_tc_tiling_on_sc=False),
        interpret=interpret)
    def sck(src_hbm, dst_hbm, sd_hbm, *rest):
        ht = rest[:nheads]
        outs = rest[nheads:2 * nheads]
        (src_v, dst_v, s_v, d_v, ex_v, rows_v0, rows_v1, st_v0, st_v1,
         zeros_v, acc_sh, gsem0, gsem1, zsem, ssem0, ssem1) = rest[2 * nheads:]
        cid = lax.axis_index("c")
        sid = lax.axis_index("s")
        start = jnp.where(cid == 0, sid * NCH0, NCH0 * NS + sid * NCH1)
        nch = jnp.where(cid == 0, NCH0, NCH1)
        pltpu.sync_copy(src_hbm.at[pl.ds(start, NCHMAX)], src_v)
        pltpu.sync_copy(dst_hbm.at[pl.ds(start, NCHMAX)], dst_v)
        z16 = jnp.zeros((16,), jnp.float32)
        for r in range(ZR):
            for q in range(ACOL // 16):
                zeros_v[r, pl.ds(q * 16, 16)] = z16
        iota16 = lax.iota(jnp.int32, 16)
        lanes = [jnp.full((16,), l, jnp.int32) for l in range(16)]
        row0 = sid * ROWS_PER_TILE

        def compute_ex(c):
            @plsc.parallel_loop(0, K // 16)
            def _exbody(j):
                si = src_v[c, pl.ds(j * 16, 16)]
                di = dst_v[c, pl.ds(j * 16, 16)]
                sv = plsc.load_gather(s_v, [si])
                dv = plsc.load_gather(d_v, [di])
                e = sv + dv
                e = jnp.where(e >= 0.0, e, e * NEG_SLOPE)
                ex_v[pl.ds(j * 16, 16)] = jnp.exp(e)

        def scale(rows_v, st_v):
            @plsc.parallel_loop(0, K // 16)
            def sbody(g):
                ex16 = ex_v[pl.ds(g * 16, 16)]
                base = g * 16
                for l in range(16):
                    av = ex16.at[lanes[l]].get(mode="promise_in_bounds")
                    j = base + l
                    for q in range(NHID // 16):
                        st_v[j, pl.ds(q * 16, 16)] = (
                            rows_v[j, pl.ds(q * 16, 16)] * av)
                    st_v[j, pl.ds(NHID, 16)] = jnp.where(
                        iota16 == 0, av, 0.0)

        for h in range(nheads):
            pltpu.sync_copy(sd_hbm.at[h], s_v)
            pltpu.sync_copy(sd_hbm.at[nheads + h], d_v)

            zd = [pltpu.async_copy(
                zeros_v, acc_sh.at[pl.ds(row0 + b * ZR, ZR)], zsem)
                for b in range(ROWS_PER_TILE // ZR)]
            for dsc in zd:
                dsc.wait()
            plsc.subcore_barrier()

            g0 = pltpu.async_copy(ht[h].at[src_v.at[0]], rows_v0, gsem0)

            def pair_body(i, _):
                c = 2 * i
                g1 = pltpu.async_copy(ht[h].at[src_v.at[c + 1]], rows_v1,
                                      gsem1)
                pltpu.make_async_copy(ht[h].at[src_v.at[c]], rows_v0,
                                      gsem0).wait()
                compute_ex(c)

                @pl.when(c >= 2)
                def _():
                    pltpu.make_async_copy(st_v0, acc_sh.at[dst_v.at[c]],
                                          ssem0).wait()
                scale(rows_v0, st_v0)
                pltpu.async_copy(st_v0, acc_sh.at[dst_v.at[c]], ssem0,
                                 add=True)

                @pl.when(c + 2 < nch)
                def _():
                    pltpu.async_copy(ht[h].at[src_v.at[c + 2]], rows_v0,
                                     gsem0)
                g1.wait()
                compute_ex(c + 1)

                @pl.when(c >= 2)
                def _():
                    pltpu.make_async_copy(st_v1, acc_sh.at[dst_v.at[c + 1]],
                                          ssem1).wait()
                scale(rows_v1, st_v1)
                pltpu.async_copy(st_v1, acc_sh.at[dst_v.at[c + 1]], ssem1,
                                 add=True)
                return 0
            lax.fori_loop(0, nch // 2, pair_body, 0)
            pltpu.make_async_copy(st_v0, acc_sh.at[dst_v.at[nch - 2]],
                                  ssem0).wait()
            pltpu.make_async_copy(st_v1, acc_sh.at[dst_v.at[nch - 1]],
                                  ssem1).wait()
            plsc.subcore_barrier()
            pltpu.sync_copy(
                acc_sh.at[pl.ds(row0, ROWS_PER_TILE)],
                outs[h].at[cid, pl.ds(row0, ROWS_PER_TILE)])
            plsc.subcore_barrier()

    return sck


_tc1 = _make_tc1()
_tc2 = _make_tc_mid(double_elu=True)
_tc3 = _make_tc3()
_tc4 = _make_tc4()

_SC_CACHE = {}


def _get_sc(nheads):
    if nheads not in _SC_CACHE:
        _SC_CACHE[nheads] = _make_sc(nheads)
    return _SC_CACHE[nheads]


@jax.jit
def _impl(x, edge_index, W1, a1_src, a1_dst, W2, a2_src, a2_dst,
          W3, a3_src, a3_dst):
    f32 = jnp.float32
    xp = jnp.zeros((NP, NFEAT), f32).at[:N].set(x.astype(f32))
    src = edge_index[0].astype(jnp.int32)
    dst = edge_index[1].astype(jnp.int32)
    pad = jnp.full((EPAD - E,), NP - 1, jnp.int32)
    src2 = jnp.concatenate([src, pad]).reshape(NW * NCHUNK, K)
    dst2 = jnp.concatenate([dst, pad]).reshape(NW * NCHUNK, K)
    W3p = jnp.zeros((NFEAT, NHID), f32).at[:, :NCLASS].set(W3)
    a3sp = jnp.zeros((NHID,), f32).at[:NCLASS].set(a3_src)
    a3dp = jnp.zeros((NHID,), f32).at[:NCLASS].set(a3_dst)

    sc4 = _get_sc(NHEADS)
    sc1 = _get_sc(1)
    *ht1, sd1 = _tc1(xp, W1, a1_src, a1_dst)
    acc1 = sc4(src2, dst2, sd1, *ht1)
    *ht2, sd2 = _tc2(*acc1, W2, a2_src, a2_dst)
    acc2 = sc4(src2, dst2, sd2, *ht2)
    ht3, sd3 = _tc3(*acc2, W3p, a3sp, a3dp)
    acc3 = sc1(src2, dst2, sd3, ht3)
    out = _tc4(acc3[0])
    return out[:N, :NCLASS]


def kernel(x, edge_index, n_node_features, mini_batch,
           W1, a1_src, a1_dst, W2, a2_src, a2_dst, W3, a3_src, a3_dst):
    return _impl(x, edge_index, W1, a1_src, a1_dst,
                 W2, a2_src, a2_dst, W3, a3_src, a3_dst)


# speedup vs baseline: 30.1910x; 1.0025x over previous
import functools

import jax
import jax.numpy as jnp
from jax import lax
from jax.experimental import pallas as pl
from jax.experimental.pallas import tpu as pltpu
from jax.experimental.pallas import tpu_sc as plsc

N = 10000
E = 160000
NFEAT = 256
NHID = 64
NHEADS = 4
NCLASS = 40
NEG_SLOPE = 0.2

NP = 10240
NC = 2
NS = 16
NW = NC * NS
K = 128
NCHUNK = 40
EPW = K * NCHUNK
EPAD = NW * EPW
ACOL = 80
ROWS_PER_TILE = NP // NS
ZR = 64

NCH0 = 52
NCH1 = 28
NCHMAX = max(NCH0, NCH1)

BN = 512
GRID = NP // BN


def _elu(v):
    return jnp.where(v > 0.0, v, jnp.exp(jnp.minimum(v, 0.0)) - 1.0)


def _norm_cols(acc_refs, nheads, double_elu):
    cols = []
    for h in range(nheads):
        pa = acc_refs[h][0] + acc_refs[h][1]
        o = pa[:, :NHID] / (pa[:, NHID:NHID + 1] + 1e-16)
        o = _elu(o)
        if double_elu:
            o = _elu(o)
        cols.append(o)
    return jnp.concatenate(cols, axis=1) if len(cols) > 1 else cols[0]


def _heads_matmul(xb, w_ref, as_ref, ad_ref, nheads, ht_refs, sd_ref):
    scols = []
    dcols = []
    for h in range(nheads):
        hb = jnp.dot(xb, w_ref[h], preferred_element_type=jnp.float32)
        ht_refs[h][...] = hb
        scols.append(jnp.dot(hb, as_ref[h].reshape(NHID, 1),
                             preferred_element_type=jnp.float32))
        dcols.append(jnp.dot(hb, ad_ref[h].reshape(NHID, 1),
                             preferred_element_type=jnp.float32))
    sd_ref[...] = jnp.concatenate(scols + dcols, axis=1).T


def _make_tc1(interpret=False):
    def body(x_ref, w_ref, as_ref, ad_ref, ht0, ht1, ht2, ht3, sd_ref):
        _heads_matmul(x_ref[...], w_ref, as_ref, ad_ref, NHEADS,
                      (ht0, ht1, ht2, ht3), sd_ref)

    full = lambda shape: pl.BlockSpec(shape, lambda i: (0,) * len(shape))
    return pl.pallas_call(
        body,
        grid=(GRID,),
        in_specs=[
            pl.BlockSpec((BN, NFEAT), lambda i: (i, 0)),
            full((NHEADS, NFEAT, NHID)),
            full((NHEADS, NHID)),
            full((NHEADS, NHID)),
        ],
        out_specs=[pl.BlockSpec((BN, NHID), lambda i: (i, 0))] * NHEADS
        + [pl.BlockSpec((2 * NHEADS, BN), lambda i: (0, i))],
        out_shape=[jax.ShapeDtypeStruct((NP, NHID), jnp.float32)] * NHEADS
        + [jax.ShapeDtypeStruct((2 * NHEADS, NP), jnp.float32)],
        interpret=interpret,
    )


def _make_tc_mid(double_elu, interpret=False):
    def body(a0, a1, a2, a3, w_ref, as_ref, ad_ref, ht0, ht1, ht2, ht3, sd_ref):
        hcat = _norm_cols((a0, a1, a2, a3), NHEADS, double_elu)
        _heads_matmul(hcat, w_ref, as_ref, ad_ref, NHEADS,
                      (ht0, ht1, ht2, ht3), sd_ref)

    full = lambda shape: pl.BlockSpec(shape, lambda i: (0,) * len(shape))
    return pl.pallas_call(
        body,
        grid=(GRID,),
        in_specs=[pl.BlockSpec((NC, BN, ACOL), lambda i: (0, i, 0))] * NHEADS
        + [
            full((NHEADS, NFEAT, NHID)),
            full((NHEADS, NHID)),
            full((NHEADS, NHID)),
        ],
        out_specs=[pl.BlockSpec((BN, NHID), lambda i: (i, 0))] * NHEADS
        + [pl.BlockSpec((2 * NHEADS, BN), lambda i: (0, i))],
        out_shape=[jax.ShapeDtypeStruct((NP, NHID), jnp.float32)] * NHEADS
        + [jax.ShapeDtypeStruct((2 * NHEADS, NP), jnp.float32)],
        interpret=interpret,
    )


def _make_tc3(interpret=False):
    def body(a0, a1, a2, a3, w_ref, as_ref, ad_ref, ht_ref, sd_ref):
        hcat = _norm_cols((a0, a1, a2, a3), NHEADS, False)
        hb = jnp.dot(hcat, w_ref[...], preferred_element_type=jnp.float32)
        ht_ref[...] = hb
        s = jnp.dot(hb, as_ref[...].reshape(NHID, 1),
                    preferred_element_type=jnp.float32)
        d = jnp.dot(hb, ad_ref[...].reshape(NHID, 1),
                    preferred_element_type=jnp.float32)
        sd_ref[...] = jnp.concatenate([s, d], axis=1).T

    full = lambda shape: pl.BlockSpec(shape, lambda i: (0,) * len(shape))
    return pl.pallas_call(
        body,
        grid=(GRID,),
        in_specs=[pl.BlockSpec((NC, BN, ACOL), lambda i: (0, i, 0))] * NHEADS
        + [full((NFEAT, NHID)), full((NHID,)), full((NHID,))],
        out_specs=[
            pl.BlockSpec((BN, NHID), lambda i: (i, 0)),
            pl.BlockSpec((2, BN), lambda i: (0, i)),
        ],
        out_shape=[
            jax.ShapeDtypeStruct((NP, NHID), jnp.float32),
            jax.ShapeDtypeStruct((2, NP), jnp.float32),
        ],
        interpret=interpret,
    )


def _make_tc4(interpret=False):
    def body(a_ref, o_ref):
        pa = a_ref[0] + a_ref[1]
        o = pa[:, :NHID] / (pa[:, NHID:NHID + 1] + 1e-16)
        o_ref[...] = _elu(o)

    return pl.pallas_call(
        body,
        grid=(GRID,),
        in_specs=[pl.BlockSpec((NC, BN, ACOL), lambda i: (0, i, 0))],
        out_specs=pl.BlockSpec((BN, NHID), lambda i: (i, 0)),
        out_shape=jax.ShapeDtypeStruct((NP, NHID), jnp.float32),
        interpret=interpret,
    )


def _make_sc(nheads, interpret=False):
    mesh = plsc.VectorSubcoreMesh(core_axis_name="c", subcore_axis_name="s",
                                  num_cores=NC, num_subcores=NS)
    out_type = [jax.ShapeDtypeStruct((NC, NP, ACOL), jnp.float32)
                for _ in range(nheads)]
    scratch = [
        pltpu.VMEM((NCHMAX, K), jnp.int32),
        pltpu.VMEM((NCHMAX, K), jnp.int32),
        pltpu.VMEM((NP,), jnp.float32),
        pltpu.VMEM((NP,), jnp.float32),
        pltpu.VMEM((K,), jnp.float32),
        pltpu.VMEM((K, NHID), jnp.float32),
        pltpu.VMEM((K, NHID), jnp.float32),
        pltpu.VMEM((K, ACOL), jnp.float32),
        pltpu.VMEM((K, ACOL), jnp.float32),
        pltpu.VMEM((ZR, ACOL), jnp.float32),
        pltpu.VMEM_SHARED((NP, ACOL), jnp.float32),
        pltpu.SemaphoreType.DMA,
        pltpu.SemaphoreType.DMA,
        pltpu.SemaphoreType.DMA,
        pltpu.SemaphoreType.DMA,
        pltpu.SemaphoreType.DMA,
    ]

    @functools.partial(
        pl.kernel, out_type=out_type, mesh=mesh, scratch_types=scratch,
        compiler_params=pltpu.CompilerParams(needs_layout_passes=False,
                                             use_tc_tiling_on_sc=False),
        interpret=interpret)
    def sck(src_hbm, dst_hbm, sd_hbm, *rest):
        ht = rest[:nheads]
        outs = rest[nheads:2 * nheads]
        (src_v, dst_v, s_v, d_v, ex_v, rows_v0, rows_v1, st_v0, st_v1,
         zeros_v, acc_sh, gsem0, gsem1, zsem, ssem0, ssem1) = rest[2 * nheads:]
        cid = lax.axis_index("c")
        sid = lax.axis_index("s")
        start = jnp.where(cid == 0, sid * NCH0, NCH0 * NS + sid * NCH1)
        nch = jnp.where(cid == 0, NCH0, NCH1)
        pltpu.sync_copy(src_hbm.at[pl.ds(start, NCHMAX)], src_v)
        pltpu.sync_copy(dst_hbm.at[pl.ds(start, NCHMAX)], dst_v)
        z16 = jnp.zeros((16,), jnp.float32)
        for r in range(ZR):
            for q in range(ACOL // 16):
                zeros_v[r, pl.ds(q * 16, 16)] = z16
        iota16 = lax.iota(jnp.int32, 16)
        lanes = [jnp.full((16,), l, jnp.int32) for l in range(16)]
        row0 = sid * ROWS_PER_TILE

        def compute_ex(c):
            @plsc.parallel_loop(0, K // 16)
            def _exbody(j):
                si = src_v[c, pl.ds(j * 16, 16)]
                di = dst_v[c, pl.ds(j * 16, 16)]
                sv = plsc.load_gather(s_v, [si])
                dv = plsc.load_gather(d_v, [di])
                e = sv + dv
                e = jnp.where(e >= 0.0, e, e * NEG_SLOPE)
                ex_v[pl.ds(j * 16, 16)] = jnp.exp(e)

        def scale(rows_v, st_v):
            @plsc.parallel_loop(0, K // 16)
            def sbody(g):
                ex16 = ex_v[pl.ds(g * 16, 16)]
                base = g * 16
                for l in range(16):
                    av = ex16.at[lanes[l]].get(mode="promise_in_bounds")
                    j = base + l
                    for q in range(NHID // 16):
                        st_v[j, pl.ds(q * 16, 16)] = (
                            rows_v[j, pl.ds(q * 16, 16)] * av)
                    st_v[j, pl.ds(NHID, 16)] = jnp.where(
                        iota16 == 0, av, 0.0)

        for h in range(nheads):
            pltpu.sync_copy(sd_hbm.at[h], s_v)
            pltpu.sync_copy(sd_hbm.at[nheads + h], d_v)

            zd = [pltpu.async_copy(
                zeros_v, acc_sh.at[pl.ds(row0 + b * ZR, ZR)], zsem)
                for b in range(ROWS_PER_TILE // ZR)]
            for dsc in zd:
                dsc.wait()
            plsc.subcore_barrier()

            g0 = pltpu.async_copy(ht[h].at[src_v.at[0]], rows_v0, gsem0)

            def pair_body(i, _):
                c = 2 * i
                g1 = pltpu.async_copy(ht[h].at[src_v.at[c + 1]], rows_v1,
                                      gsem1)
                pltpu.make_async_copy(ht[h].at[src_v.at[c]], rows_v0,
                                      gsem0).wait()
                compute_ex(c)

                @pl.when(c >= 2)
                def _():
                    pltpu.make_async_copy(st_v0, acc_sh.at[dst_v.at[c]],
                                          ssem0).wait()
                scale(rows_v0, st_v0)
                pltpu.async_copy(st_v0, acc_sh.at[dst_v.at[c]], ssem0,
                                 add=True)

                @pl.when(c + 2 < nch)
                def _():
                    pltpu.async_copy(ht[h].at[src_v.at[c + 2]], rows_v0,
                                     gsem0)
                g1.wait()
                compute_ex(c + 1)

                @pl.when(c >= 2)
                def _():
                    pltpu.make_async_copy(st_v1, acc_sh.at[dst_v.at[c + 1]],
                                          ssem1).wait()
                scale(rows_v1, st_v1)
                pltpu.async_copy(st_v1, acc_sh.at[dst_v.at[c + 1]], ssem1,
                                 add=True)
                return 0
            lax.fori_loop(0, nch // 2, pair_body, 0)
            pltpu.make_async_copy(st_v0, acc_sh.at[dst_v.at[nch - 2]],
                                  ssem0).wait()
            pltpu.make_async_copy(st_v1, acc_sh.at[dst_v.at[nch - 1]],
                                  ssem1).wait()
            plsc.subcore_barrier()
            pltpu.sync_copy(
                acc_sh.at[pl.ds(row0, ROWS_PER_TILE)],
                outs[h].at[cid, pl.ds(row0, ROWS_PER_TILE)])
            plsc.subcore_barrier()

    return sck


_tc1 = _make_tc1()
_tc2 = _make_tc_mid(double_elu=True)
_tc3 = _make_tc3()
_tc4 = _make_tc4()

_SC_CACHE = {}


def _get_sc(nheads):
    if nheads not in _SC_CACHE:
        _SC_CACHE[nheads] = _make_sc(nheads)
    return _SC_CACHE[nheads]


@jax.jit
def _impl(x, edge_index, W1, a1_src, a1_dst, W2, a2_src, a2_dst,
          W3, a3_src, a3_dst):
    f32 = jnp.float32
    xp = jnp.zeros((NP, NFEAT), f32).at[:N].set(x.astype(f32))
    src = edge_index[0].astype(jnp.int32)
    dst = edge_index[1].astype(jnp.int32)
    pad = jnp.full((EPAD - E,), NP - 1, jnp.int32)
    src2 = jnp.concatenate([src, pad]).reshape(NW * NCHUNK, K)
    dst2 = jnp.concatenate([dst, pad]).reshape(NW * NCHUNK, K)
    W3p = jnp.zeros((NFEAT, NHID), f32).at[:, :NCLASS].set(W3)
    a3sp = jnp.zeros((NHID,), f32).at[:NCLASS].set(a3_src)
    a3dp = jnp.zeros((NHID,), f32).at[:NCLASS].set(a3_dst)

    sc4 = _get_sc(NHEADS)
    sc1 = _get_sc(1)
    *ht1, sd1 = _tc1(xp, W1, a1_src, a1_dst)
    acc1 = sc4(src2, dst2, sd1, *ht1)
    *ht2, sd2 = _tc2(*acc1, W2, a2_src, a2_dst)
    acc2 = sc4(src2, dst2, sd2, *ht2)
    ht3, sd3 = _tc3(*acc2, W3p, a3sp, a3dp)
    acc3 = sc1(src2, dst2, sd3, ht3)
    out = _tc4(acc3[0])
    return out[:N, :NCLASS]


def kernel(x, edge_index, n_node_features, mini_batch,
           W1, a1_src, a1_dst, W2, a2_src, a2_dst, W3, a3_src, a3_dst):
    return _impl(x, edge_index, W1, a1_src, a1_dst,
                 W2, a2_src, a2_dst, W3, a3_src, a3_dst)
